# Initial kernel scaffold; baseline (speedup 1.0000x reference)
import functools

import jax
import jax.numpy as jnp
from jax import lax
from jax.experimental import pallas as pl
from jax.experimental.pallas import tpu as pltpu
from jax.experimental.pallas import tpu_sc as plsc

NC = 2
NS = 16
NW = NC * NS
CHUNK = 128
DH = 16


def _round_up(a, b):
    return (a + b - 1) // b * b


def _mm1_body(x_ref, w_ref, o_ref):
    o_ref[...] = jnp.dot(x_ref[...], w_ref[...],
                         preferred_element_type=jnp.float32)


def _tc_matmul(x_p, W1):
    npad = x_p.shape[0]
    return pl.pallas_call(
        _mm1_body,
        out_shape=jax.ShapeDtypeStruct((npad, DH), jnp.float32),
    )(x_p, W1)


def _scale1_body(deg_ref, h1_ref, dis_ref, g1_ref):
    deg = deg_ref[0, :] + deg_ref[1, :] + 1.0
    dis = lax.rsqrt(deg)
    dis_ref[...] = dis
    g1_ref[...] = dis[:, None] * h1_ref[...]


def _tc_scale1(degp, h1):
    npad = h1.shape[0]
    return pl.pallas_call(
        _scale1_body,
        out_shape=[
            jax.ShapeDtypeStruct((npad,), jnp.float32),
            jax.ShapeDtypeStruct((npad, DH), jnp.float32),
        ],
    )(degp, h1)


def _mid_body(s_ref, g1_ref, dis_ref, b1_ref, g2_ref):
    tot = s_ref[0] + s_ref[1] + g1_ref[...]
    agg = dis_ref[...][:, None] * tot + b1_ref[...][None, :]
    r = jnp.maximum(agg, 0.0)
    g2_ref[...] = dis_ref[...][:, None] * r


def _tc_mid(S1, g1, dis, b1):
    npad = g1.shape[0]
    return pl.pallas_call(
        _mid_body,
        out_shape=jax.ShapeDtypeStruct((npad, DH), jnp.float32),
    )(S1, g1, dis, b1)


def _final_body(s_ref, g2_ref, dis_ref, w2_ref, b2_ref, o_ref):
    tot = s_ref[0] + s_ref[1] + g2_ref[...]
    agg = dis_ref[...][:, None] * tot
    z = jnp.dot(agg, w2_ref[...], preferred_element_type=jnp.float32)
    z = z + b2_ref[...][None, :]
    m = jnp.max(z, axis=1, keepdims=True)
    lse = m + jnp.log(jnp.sum(jnp.exp(z - m), axis=1, keepdims=True))
    o_ref[...] = z - lse


def _tc_final(S2, g2, dis, W2, b2):
    npad = g2.shape[0]
    dout = W2.shape[1]
    return pl.pallas_call(
        _final_body,
        out_shape=jax.ShapeDtypeStruct((npad, dout), jnp.float32),
    )(S2, g2, dis, W2, b2)


def _sc_mesh():
    return plsc.VectorSubcoreMesh(core_axis_name="c", subcore_axis_name="s",
                                  num_cores=NC, num_subcores=NS)


def _sc_degree(dstm, npad):
    k_per_w = dstm.shape[0] // NW
    zr = npad // NS

    @functools.partial(
        pl.kernel,
        out_type=jax.ShapeDtypeStruct((NC, npad), jnp.float32),
        mesh=_sc_mesh(),
        scratch_types=[
            pltpu.VMEM((k_per_w, CHUNK), jnp.int32),
            pltpu.VMEM((CHUNK,), jnp.float32),
            pltpu.VMEM_SHARED((npad,), jnp.float32),
            pltpu.SemaphoreType.DMA,
        ],
    )
    def k(dst_hbm, out_hbm, dst_v, ones_v, acc_sh, sem):
        c = lax.axis_index("c")
        s = lax.axis_index("s")
        wid = c * NS + s

        @pl.loop(0, CHUNK, step=16)
        def _(i):
            ones_v[pl.ds(i, 16)] = jnp.zeros((16,), jnp.float32)

        @pl.loop(0, zr, step=CHUNK)
        def _(r):
            pltpu.sync_copy(ones_v, acc_sh.at[pl.ds(s * zr + r, CHUNK)])

        pltpu.sync_copy(dst_hbm.at[pl.ds(wid * k_per_w, k_per_w)], dst_v)

        @pl.loop(0, CHUNK, step=16)
        def _(i):
            ones_v[pl.ds(i, 16)] = jnp.ones((16,), jnp.float32)

        plsc.subcore_barrier()

        @pl.loop(0, k_per_w)
        def _(j):
            pltpu.sync_copy(ones_v, acc_sh.at[dst_v.at[j]], add=True)

        plsc.subcore_barrier()

        @pl.loop(0, zr, step=CHUNK)
        def _(r):
            pltpu.sync_copy(acc_sh.at[pl.ds(s * zr + r, CHUNK)],
                            out_hbm.at[c, pl.ds(s * zr + r, CHUNK)])

    return k(dstm)


def _sc_scatter(g, srcm, dstm):
    npad = g.shape[0]
    k_per_w = srcm.shape[0] // NW
    zr = npad // NS

    @functools.partial(
        pl.kernel,
        out_type=jax.ShapeDtypeStruct((NC, npad, DH), jnp.float32),
        mesh=_sc_mesh(),
        scratch_types=[
            pltpu.VMEM((k_per_w, CHUNK), jnp.int32),
            pltpu.VMEM((k_per_w, CHUNK), jnp.int32),
            pltpu.VMEM((CHUNK, DH), jnp.float32),
            pltpu.VMEM_SHARED((npad, DH), jnp.float32),
            pltpu.SemaphoreType.DMA,
        ],
    )
    def k(g_hbm, src_hbm, dst_hbm, out_hbm, src_v, dst_v, rows_v, acc_sh, sem):
        c = lax.axis_index("c")
        s = lax.axis_index("s")
        wid = c * NS + s

        @pl.loop(0, CHUNK)
        def _(i):
            rows_v[i] = jnp.zeros((DH,), jnp.float32)

        @pl.loop(0, zr, step=CHUNK)
        def _(r):
            pltpu.sync_copy(rows_v, acc_sh.at[pl.ds(s * zr + r, CHUNK)])

        pltpu.sync_copy(src_hbm.at[pl.ds(wid * k_per_w, k_per_w)], src_v)
        pltpu.sync_copy(dst_hbm.at[pl.ds(wid * k_per_w, k_per_w)], dst_v)

        plsc.subcore_barrier()

        @pl.loop(0, k_per_w)
        def _(j):
            pltpu.async_copy(g_hbm.at[src_v.at[j]], rows_v, sem).wait()
            pltpu.sync_copy(rows_v, acc_sh.at[dst_v.at[j]], add=True)

        plsc.subcore_barrier()

        @pl.loop(0, zr, step=CHUNK)
        def _(r):
            pltpu.sync_copy(acc_sh.at[pl.ds(s * zr + r, CHUNK)],
                            out_hbm.at[c, pl.ds(s * zr + r, CHUNK)])

    return k(g, srcm, dstm)


def kernel(x, edge_index, W1, b1, W2, b2):
    n, _ = x.shape
    e = edge_index.shape[1]
    npad = _round_up(n + 1, NS * CHUNK)
    epad = _round_up(e, NW * CHUNK)

    src = edge_index[0].astype(jnp.int32)
    dst = edge_index[1].astype(jnp.int32)
    pad = jnp.full((epad - e,), n, jnp.int32)
    srcm = jnp.concatenate([src, pad]).reshape(-1, CHUNK)
    dstm = jnp.concatenate([dst, pad]).reshape(-1, CHUNK)
    x_p = jnp.zeros((npad, x.shape[1]), x.dtype).at[:n].set(x)

    h1 = _tc_matmul(x_p, W1)
    degp = _sc_degree(dstm, npad)
    dis, g1 = _tc_scale1(degp, h1)
    S1 = _sc_scatter(g1, srcm, dstm)
    g2 = _tc_mid(S1, g1, dis, b1)
    S2 = _sc_scatter(g2, srcm, dstm)
    out = _tc_final(S2, g2, dis, W2, b2)
    return out[:n]


# speedup vs baseline: 32.4035x; 32.4035x over previous
import functools

import jax
import jax.numpy as jnp
from jax import lax
from jax.experimental import pallas as pl
from jax.experimental.pallas import tpu as pltpu
from jax.experimental.pallas import tpu_sc as plsc

NC = 2
NS = 16
NW = NC * NS
CHUNK = 128
DH = 16


def _round_up(a, b):
    return (a + b - 1) // b * b


def _mm1_body(x_ref, w_ref, o_ref):
    o_ref[...] = jnp.dot(x_ref[...], w_ref[...],
                         preferred_element_type=jnp.float32)


def _tc_matmul(x_p, W1):
    npad = x_p.shape[0]
    return pl.pallas_call(
        _mm1_body,
        out_shape=jax.ShapeDtypeStruct((npad, DH), jnp.float32),
    )(x_p, W1)


def _scale1_body(deg_ref, h1_ref, dis_ref, g1_ref):
    deg = deg_ref[0, :] + deg_ref[1, :] + 1.0
    dis = lax.rsqrt(deg)
    dis_ref[...] = dis
    g1_ref[...] = dis[:, None] * h1_ref[...]


def _tc_scale1(degp, h1):
    npad = h1.shape[0]
    return pl.pallas_call(
        _scale1_body,
        out_shape=[
            jax.ShapeDtypeStruct((npad,), jnp.float32),
            jax.ShapeDtypeStruct((npad, DH), jnp.float32),
        ],
    )(degp, h1)


def _mid_body(s_ref, g1_ref, dis_ref, b1_ref, g2_ref):
    tot = s_ref[0] + s_ref[1] + g1_ref[...]
    agg = dis_ref[...][:, None] * tot + b1_ref[...][None, :]
    r = jnp.maximum(agg, 0.0)
    g2_ref[...] = dis_ref[...][:, None] * r


def _tc_mid(S1, g1, dis, b1):
    npad = g1.shape[0]
    return pl.pallas_call(
        _mid_body,
        out_shape=jax.ShapeDtypeStruct((npad, DH), jnp.float32),
    )(S1, g1, dis, b1)


def _final_body(s_ref, g2_ref, dis_ref, w2_ref, b2_ref, o_ref):
    tot = s_ref[0] + s_ref[1] + g2_ref[...]
    agg = dis_ref[...][:, None] * tot
    z = jnp.dot(agg, w2_ref[...], preferred_element_type=jnp.float32)
    z = z + b2_ref[...][None, :]
    m = jnp.max(z, axis=1, keepdims=True)
    lse = m + jnp.log(jnp.sum(jnp.exp(z - m), axis=1, keepdims=True))
    o_ref[...] = z - lse


def _tc_final(S2, g2, dis, W2, b2):
    npad = g2.shape[0]
    dout = W2.shape[1]
    return pl.pallas_call(
        _final_body,
        out_shape=jax.ShapeDtypeStruct((npad, dout), jnp.float32),
    )(S2, g2, dis, W2, b2)


def _sc_mesh():
    return plsc.VectorSubcoreMesh(core_axis_name="c", subcore_axis_name="s",
                                  num_cores=NC, num_subcores=NS)


_SC_PARAMS = pltpu.CompilerParams(use_tc_tiling_on_sc=False)


def _sc_degree(dstm, npad):
    k_per_w = dstm.shape[0] // NW
    zr = npad // NS

    @functools.partial(
        pl.kernel,
        out_type=jax.ShapeDtypeStruct((NC, npad), jnp.float32),
        mesh=_sc_mesh(),
        scratch_types=[
            pltpu.VMEM((k_per_w, CHUNK), jnp.int32),
            pltpu.VMEM((CHUNK,), jnp.float32),
            pltpu.VMEM_SHARED((npad,), jnp.float32),
            pltpu.SemaphoreType.DMA,
        ],
        compiler_params=_SC_PARAMS,
    )
    def k(dst_hbm, out_hbm, dst_v, ones_v, acc_sh, sem):
        c = lax.axis_index("c")
        s = lax.axis_index("s")
        wid = c * NS + s

        @pl.loop(0, CHUNK, step=16)
        def _(i):
            ones_v[pl.ds(i, 16)] = jnp.zeros((16,), jnp.float32)

        @pl.loop(0, zr, step=CHUNK)
        def _(r):
            pltpu.sync_copy(ones_v, acc_sh.at[pl.ds(s * zr + r, CHUNK)])

        pltpu.sync_copy(dst_hbm.at[pl.ds(wid * k_per_w, k_per_w)], dst_v)

        @pl.loop(0, CHUNK, step=16)
        def _(i):
            ones_v[pl.ds(i, 16)] = jnp.ones((16,), jnp.float32)

        plsc.subcore_barrier()

        @pl.loop(0, k_per_w)
        def _(j):
            pltpu.sync_copy(ones_v, acc_sh.at[dst_v.at[j]], add=True)

        plsc.subcore_barrier()

        @pl.loop(0, zr, step=CHUNK)
        def _(r):
            pltpu.sync_copy(acc_sh.at[pl.ds(s * zr + r, CHUNK)],
                            out_hbm.at[c, pl.ds(s * zr + r, CHUNK)])

    return k(dstm)


def _sc_scatter(g, srcm, dstm):
    npad = g.shape[0]
    k_per_w = srcm.shape[0] // NW
    zr = npad // NS

    @functools.partial(
        pl.kernel,
        out_type=jax.ShapeDtypeStruct((NC, npad, DH), jnp.float32),
        mesh=_sc_mesh(),
        scratch_types=[
            pltpu.VMEM((k_per_w, CHUNK), jnp.int32),
            pltpu.VMEM((k_per_w, CHUNK), jnp.int32),
            pltpu.VMEM((CHUNK, DH), jnp.float32),
            pltpu.VMEM_SHARED((npad, DH), jnp.float32),
            pltpu.SemaphoreType.DMA,
        ],
        compiler_params=_SC_PARAMS,
    )
    def k(g_hbm, src_hbm, dst_hbm, out_hbm, src_v, dst_v, rows_v, acc_sh, sem):
        c = lax.axis_index("c")
        s = lax.axis_index("s")
        wid = c * NS + s

        @pl.loop(0, CHUNK)
        def _(i):
            rows_v[i] = jnp.zeros((DH,), jnp.float32)

        @pl.loop(0, zr, step=CHUNK)
        def _(r):
            pltpu.sync_copy(rows_v, acc_sh.at[pl.ds(s * zr + r, CHUNK)])

        pltpu.sync_copy(src_hbm.at[pl.ds(wid * k_per_w, k_per_w)], src_v)
        pltpu.sync_copy(dst_hbm.at[pl.ds(wid * k_per_w, k_per_w)], dst_v)

        plsc.subcore_barrier()

        @pl.loop(0, k_per_w)
        def _(j):
            pltpu.async_copy(g_hbm.at[src_v.at[j]], rows_v, sem).wait()
            pltpu.sync_copy(rows_v, acc_sh.at[dst_v.at[j]], add=True)

        plsc.subcore_barrier()

        @pl.loop(0, zr, step=CHUNK)
        def _(r):
            pltpu.sync_copy(acc_sh.at[pl.ds(s * zr + r, CHUNK)],
                            out_hbm.at[c, pl.ds(s * zr + r, CHUNK)])

    return k(g, srcm, dstm)


def kernel(x, edge_index, W1, b1, W2, b2):
    n, _ = x.shape
    e = edge_index.shape[1]
    npad = _round_up(n + 1, NS * CHUNK)
    epad = _round_up(e, NW * CHUNK * 8)

    src = edge_index[0].astype(jnp.int32)
    dst = edge_index[1].astype(jnp.int32)
    pad = jnp.full((epad - e,), n, jnp.int32)
    srcm = jnp.concatenate([src, pad]).reshape(-1, CHUNK)
    dstm = jnp.concatenate([dst, pad]).reshape(-1, CHUNK)
    x_p = jnp.zeros((npad, x.shape[1]), x.dtype).at[:n].set(x)

    h1 = _tc_matmul(x_p, W1)
    degp = _sc_degree(dstm, npad)
    dis, g1 = _tc_scale1(degp, h1)
    S1 = _sc_scatter(g1, srcm, dstm)
    g2 = _tc_mid(S1, g1, dis, b1)
    S2 = _sc_scatter(g2, srcm, dstm)
    out = _tc_final(S2, g2, dis, W2, b2)
    return out[:n]


# speedup vs baseline: 43.6970x; 1.3485x over previous
import functools

import jax
import jax.numpy as jnp
from jax import lax
from jax.experimental import pallas as pl
from jax.experimental.pallas import tpu as pltpu
from jax.experimental.pallas import tpu_sc as plsc

NC = 2
NS = 16
NW = NC * NS
CHUNK = 128
DH = 16


def _round_up(a, b):
    return (a + b - 1) // b * b


def _mm1_body(x_ref, w_ref, o_ref):
    o_ref[...] = jnp.dot(x_ref[...], w_ref[...],
                         preferred_element_type=jnp.float32)


def _tc_matmul(x_p, W1):
    npad = x_p.shape[0]
    return pl.pallas_call(
        _mm1_body,
        out_shape=jax.ShapeDtypeStruct((npad, DH), jnp.float32),
    )(x_p, W1)


def _scale1_body(deg_ref, h1_ref, dis_ref, g1_ref):
    deg = deg_ref[0, :] + deg_ref[1, :] + 1.0
    dis = lax.rsqrt(deg)
    dis_ref[...] = dis
    g1_ref[...] = dis[:, None] * h1_ref[...]


def _tc_scale1(degp, h1):
    npad = h1.shape[0]
    return pl.pallas_call(
        _scale1_body,
        out_shape=[
            jax.ShapeDtypeStruct((npad,), jnp.float32),
            jax.ShapeDtypeStruct((npad, DH), jnp.float32),
        ],
    )(degp, h1)


def _mid_body(s_ref, g1_ref, dis_ref, b1_ref, g2_ref):
    tot = s_ref[0] + s_ref[1] + g1_ref[...]
    agg = dis_ref[...][:, None] * tot + b1_ref[...][None, :]
    r = jnp.maximum(agg, 0.0)
    g2_ref[...] = dis_ref[...][:, None] * r


def _tc_mid(S1, g1, dis, b1):
    npad = g1.shape[0]
    return pl.pallas_call(
        _mid_body,
        out_shape=jax.ShapeDtypeStruct((npad, DH), jnp.float32),
    )(S1, g1, dis, b1)


def _final_body(s_ref, g2_ref, dis_ref, w2_ref, b2_ref, o_ref):
    tot = s_ref[0] + s_ref[1] + g2_ref[...]
    agg = dis_ref[...][:, None] * tot
    z = jnp.dot(agg, w2_ref[...], preferred_element_type=jnp.float32)
    z = z + b2_ref[...][None, :]
    m = jnp.max(z, axis=1, keepdims=True)
    lse = m + jnp.log(jnp.sum(jnp.exp(z - m), axis=1, keepdims=True))
    o_ref[...] = z - lse


def _tc_final(S2, g2, dis, W2, b2):
    npad = g2.shape[0]
    dout = W2.shape[1]
    return pl.pallas_call(
        _final_body,
        out_shape=jax.ShapeDtypeStruct((npad, dout), jnp.float32),
    )(S2, g2, dis, W2, b2)


def _sc_mesh():
    return plsc.VectorSubcoreMesh(core_axis_name="c", subcore_axis_name="s",
                                  num_cores=NC, num_subcores=NS)


_SC_PARAMS = pltpu.CompilerParams(use_tc_tiling_on_sc=False)


def _sc_degree(dstm, npad):
    k_per_w = dstm.shape[0] // NW
    zr = npad // NS

    @functools.partial(
        pl.kernel,
        out_type=jax.ShapeDtypeStruct((NC, npad), jnp.float32),
        mesh=_sc_mesh(),
        scratch_types=[
            pltpu.VMEM((k_per_w, CHUNK), jnp.int32),
            pltpu.VMEM((CHUNK,), jnp.float32),
            pltpu.VMEM_SHARED((npad,), jnp.float32),
            pltpu.SemaphoreType.DMA,
        ],
        compiler_params=_SC_PARAMS,
    )
    def k(dst_hbm, out_hbm, dst_v, ones_v, acc_sh, sem):
        c = lax.axis_index("c")
        s = lax.axis_index("s")
        wid = c * NS + s

        @pl.loop(0, CHUNK, step=16)
        def _(i):
            ones_v[pl.ds(i, 16)] = jnp.zeros((16,), jnp.float32)

        @pl.loop(0, zr, step=CHUNK)
        def _(r):
            pltpu.sync_copy(ones_v, acc_sh.at[pl.ds(s * zr + r, CHUNK)])

        pltpu.sync_copy(dst_hbm.at[pl.ds(wid * k_per_w, k_per_w)], dst_v)

        @pl.loop(0, CHUNK, step=16)
        def _(i):
            ones_v[pl.ds(i, 16)] = jnp.ones((16,), jnp.float32)

        plsc.subcore_barrier()

        @pl.loop(0, k_per_w)
        def _(j):
            pltpu.sync_copy(ones_v, acc_sh.at[dst_v.at[j]], add=True)

        plsc.subcore_barrier()

        @pl.loop(0, zr, step=CHUNK)
        def _(r):
            pltpu.sync_copy(acc_sh.at[pl.ds(s * zr + r, CHUNK)],
                            out_hbm.at[c, pl.ds(s * zr + r, CHUNK)])

    return k(dstm)


def _sc_scatter(g, srcm, dstm):
    npad = g.shape[0]
    k_per_w = srcm.shape[0] // NW
    zr = npad // NS

    nb = 4
    assert k_per_w % (2 * nb) == 0 and k_per_w >= 4 * nb

    @functools.partial(
        pl.kernel,
        out_type=jax.ShapeDtypeStruct((NC, npad, DH), jnp.float32),
        mesh=_sc_mesh(),
        scratch_types=[
            pltpu.VMEM((k_per_w, CHUNK), jnp.int32),
            pltpu.VMEM((k_per_w, CHUNK), jnp.int32),
        ] + [pltpu.VMEM((CHUNK, DH), jnp.float32)] * (2 * nb) + [
            pltpu.VMEM_SHARED((npad, DH), jnp.float32),
            pltpu.SemaphoreType.DMA,
            pltpu.SemaphoreType.DMA,
        ],
        compiler_params=_SC_PARAMS,
    )
    def k(g_hbm, src_hbm, dst_hbm, out_hbm, src_v, dst_v, *rest):
        bufs_a = rest[:nb]
        bufs_b = rest[nb:2 * nb]
        acc_sh, sem_a, sem_b = rest[2 * nb:]
        c = lax.axis_index("c")
        s = lax.axis_index("s")
        wid = c * NS + s

        @pl.loop(0, CHUNK)
        def _(i):
            bufs_a[0][i] = jnp.zeros((DH,), jnp.float32)

        @pl.loop(0, zr, step=CHUNK)
        def _(r):
            pltpu.sync_copy(bufs_a[0], acc_sh.at[pl.ds(s * zr + r, CHUNK)])

        pltpu.sync_copy(src_hbm.at[pl.ds(wid * k_per_w, k_per_w)], src_v)
        pltpu.sync_copy(dst_hbm.at[pl.ds(wid * k_per_w, k_per_w)], dst_v)

        plsc.subcore_barrier()

        def gather(chunk, buf, sem):
            pltpu.async_copy(g_hbm.at[src_v.at[chunk]], buf, sem)

        def drain(buf, sem):
            pltpu.make_async_copy(g_hbm.at[src_v.at[0]], buf, sem).wait()

        def scatter(chunk, buf):
            pltpu.sync_copy(buf, acc_sh.at[dst_v.at[chunk]], add=True)

        for b in range(nb):
            gather(b, bufs_a[b], sem_a)

        @pl.loop(0, k_per_w - 2 * nb, step=2 * nb)
        def _(jv):
            for b in range(nb):
                gather(jv + nb + b, bufs_b[b], sem_b)
            for b in range(nb):
                drain(bufs_a[b], sem_a)
            for b in range(nb):
                scatter(jv + b, bufs_a[b])
            for b in range(nb):
                gather(jv + 2 * nb + b, bufs_a[b], sem_a)
            for b in range(nb):
                drain(bufs_b[b], sem_b)
            for b in range(nb):
                scatter(jv + nb + b, bufs_b[b])

        last = k_per_w - 2 * nb
        for b in range(nb):
            gather(last + nb + b, bufs_b[b], sem_b)
        for b in range(nb):
            drain(bufs_a[b], sem_a)
        for b in range(nb):
            scatter(last + b, bufs_a[b])
        for b in range(nb):
            drain(bufs_b[b], sem_b)
        for b in range(nb):
            scatter(last + nb + b, bufs_b[b])

        plsc.subcore_barrier()

        @pl.loop(0, zr, step=CHUNK)
        def _(r):
            pltpu.sync_copy(acc_sh.at[pl.ds(s * zr + r, CHUNK)],
                            out_hbm.at[c, pl.ds(s * zr + r, CHUNK)])

    return k(g, srcm, dstm)


def kernel(x, edge_index, W1, b1, W2, b2):
    n, _ = x.shape
    e = edge_index.shape[1]
    npad = _round_up(n + 1, NS * CHUNK)
    epad = _round_up(e, NW * CHUNK * 8)

    src = edge_index[0].astype(jnp.int32)
    dst = edge_index[1].astype(jnp.int32)
    pad = jnp.full((epad - e,), n, jnp.int32)
    srcm = jnp.concatenate([src, pad]).reshape(-1, CHUNK)
    dstm = jnp.concatenate([dst, pad]).reshape(-1, CHUNK)
    x_p = jnp.zeros((npad, x.shape[1]), x.dtype).at[:n].set(x)

    h1 = _tc_matmul(x_p, W1)
    degp = _sc_degree(dstm, npad)
    dis, g1 = _tc_scale1(degp, h1)
    S1 = _sc_scatter(g1, srcm, dstm)
    g2 = _tc_mid(S1, g1, dis, b1)
    S2 = _sc_scatter(g2, srcm, dstm)
    out = _tc_final(S2, g2, dis, W2, b2)
    return out[:n]


# speedup vs baseline: 59.4107x; 1.3596x over previous
import functools

import jax
import jax.numpy as jnp
from jax import lax
from jax.experimental import pallas as pl
from jax.experimental.pallas import tpu as pltpu
from jax.experimental.pallas import tpu_sc as plsc

NC = 2
NS = 16
NW = NC * NS
CHUNK = 128
DH = 16


def _round_up(a, b):
    return (a + b - 1) // b * b


def _mm1_body(x_ref, w_ref, o_ref):
    o_ref[...] = jnp.dot(x_ref[...], w_ref[...],
                         preferred_element_type=jnp.float32)


def _tc_matmul(x_p, W1):
    npad = x_p.shape[0]
    return pl.pallas_call(
        _mm1_body,
        out_shape=jax.ShapeDtypeStruct((npad, DH), jnp.float32),
    )(x_p, W1)


def _scale1_body(deg_ref, h1_ref, dis_ref, g1_ref):
    deg = deg_ref[0, :] + deg_ref[1, :] + 1.0
    dis = lax.rsqrt(deg)
    dis_ref[...] = dis
    g1_ref[...] = dis[:, None] * h1_ref[...]


def _tc_scale1(degp, h1):
    npad = h1.shape[0]
    return pl.pallas_call(
        _scale1_body,
        out_shape=[
            jax.ShapeDtypeStruct((npad,), jnp.float32),
            jax.ShapeDtypeStruct((npad, DH), jnp.float32),
        ],
    )(degp, h1)


def _mid_body(s_ref, g1_ref, dis_ref, b1_ref, g2_ref):
    tot = s_ref[0] + s_ref[1] + g1_ref[...]
    agg = dis_ref[...][:, None] * tot + b1_ref[...][None, :]
    r = jnp.maximum(agg, 0.0)
    g2_ref[...] = dis_ref[...][:, None] * r


def _tc_mid(S1, g1, dis, b1):
    npad = g1.shape[0]
    return pl.pallas_call(
        _mid_body,
        out_shape=jax.ShapeDtypeStruct((npad, DH), jnp.float32),
    )(S1, g1, dis, b1)


def _final_body(s_ref, g2_ref, dis_ref, w2_ref, b2_ref, o_ref):
    tot = s_ref[0] + s_ref[1] + g2_ref[...]
    agg = dis_ref[...][:, None] * tot
    z = jnp.dot(agg, w2_ref[...], preferred_element_type=jnp.float32)
    z = z + b2_ref[...][None, :]
    m = jnp.max(z, axis=1, keepdims=True)
    lse = m + jnp.log(jnp.sum(jnp.exp(z - m), axis=1, keepdims=True))
    o_ref[...] = z - lse


def _tc_final(S2, g2, dis, W2, b2):
    npad = g2.shape[0]
    dout = W2.shape[1]
    return pl.pallas_call(
        _final_body,
        out_shape=jax.ShapeDtypeStruct((npad, dout), jnp.float32),
    )(S2, g2, dis, W2, b2)


def _sc_mesh():
    return plsc.VectorSubcoreMesh(core_axis_name="c", subcore_axis_name="s",
                                  num_cores=NC, num_subcores=NS)


_SC_PARAMS = pltpu.CompilerParams(use_tc_tiling_on_sc=False)


def _sc_degree(dstm, npad):
    k_per_w = dstm.shape[0] // NW
    zr = npad // NS

    @functools.partial(
        pl.kernel,
        out_type=jax.ShapeDtypeStruct((NC, npad), jnp.float32),
        mesh=_sc_mesh(),
        scratch_types=[
            pltpu.VMEM((k_per_w, CHUNK), jnp.int32),
            pltpu.VMEM((CHUNK,), jnp.float32),
            pltpu.VMEM_SHARED((npad,), jnp.float32),
            pltpu.SemaphoreType.DMA,
        ],
        compiler_params=_SC_PARAMS,
    )
    def k(dst_hbm, out_hbm, dst_v, ones_v, acc_sh, sem):
        c = lax.axis_index("c")
        s = lax.axis_index("s")
        wid = c * NS + s

        @pl.loop(0, CHUNK, step=16)
        def _(i):
            ones_v[pl.ds(i, 16)] = jnp.zeros((16,), jnp.float32)

        @pl.loop(0, zr, step=CHUNK)
        def _(r):
            pltpu.sync_copy(ones_v, acc_sh.at[pl.ds(s * zr + r, CHUNK)])

        pltpu.sync_copy(dst_hbm.at[pl.ds(wid * k_per_w, k_per_w)], dst_v)

        @pl.loop(0, CHUNK, step=16)
        def _(i):
            ones_v[pl.ds(i, 16)] = jnp.ones((16,), jnp.float32)

        plsc.subcore_barrier()

        depth = 16

        @pl.loop(0, depth)
        def _(j):
            pltpu.async_copy(ones_v, acc_sh.at[dst_v.at[j]], sem, add=True)

        @pl.loop(depth, k_per_w)
        def _(j):
            pltpu.async_copy(ones_v, acc_sh.at[dst_v.at[j]], sem, add=True)
            pltpu.make_async_copy(ones_v, acc_sh.at[dst_v.at[0]], sem).wait()

        @pl.loop(0, depth)
        def _(j):
            pltpu.make_async_copy(ones_v, acc_sh.at[dst_v.at[0]], sem).wait()

        plsc.subcore_barrier()

        @pl.loop(0, zr, step=CHUNK)
        def _(r):
            pltpu.sync_copy(acc_sh.at[pl.ds(s * zr + r, CHUNK)],
                            out_hbm.at[c, pl.ds(s * zr + r, CHUNK)])

    return k(dstm)


def _sc_scatter(g, srcm, dstm):
    npad = g.shape[0]
    k_per_w = srcm.shape[0] // NW
    zr = npad // NS

    nb = 4
    assert k_per_w % (2 * nb) == 0 and k_per_w >= 4 * nb

    @functools.partial(
        pl.kernel,
        out_type=jax.ShapeDtypeStruct((NC, npad, DH), jnp.float32),
        mesh=_sc_mesh(),
        scratch_types=[
            pltpu.VMEM((k_per_w, CHUNK), jnp.int32),
            pltpu.VMEM((k_per_w, CHUNK), jnp.int32),
        ] + [pltpu.VMEM((CHUNK, DH), jnp.float32)] * (2 * nb) + [
            pltpu.VMEM_SHARED((npad, DH), jnp.float32),
            pltpu.VMEM_SHARED((npad, DH), jnp.float32),
            pltpu.SemaphoreType.DMA,
            pltpu.SemaphoreType.DMA,
        ],
        compiler_params=_SC_PARAMS,
    )
    def k(g_hbm, src_hbm, dst_hbm, out_hbm, src_v, dst_v, *rest):
        bufs_a = rest[:nb]
        bufs_b = rest[nb:2 * nb]
        acc_sh, g_sh, sem_a, sem_b = rest[2 * nb:]
        c = lax.axis_index("c")
        s = lax.axis_index("s")
        wid = c * NS + s

        @pl.loop(0, CHUNK)
        def _(i):
            bufs_a[0][i] = jnp.zeros((DH,), jnp.float32)

        @pl.loop(0, zr, step=CHUNK)
        def _(r):
            pltpu.sync_copy(bufs_a[0], acc_sh.at[pl.ds(s * zr + r, CHUNK)])

        pltpu.sync_copy(src_hbm.at[pl.ds(wid * k_per_w, k_per_w)], src_v)
        pltpu.sync_copy(dst_hbm.at[pl.ds(wid * k_per_w, k_per_w)], dst_v)
        pltpu.sync_copy(g_hbm.at[pl.ds(s * zr, zr)], g_sh.at[pl.ds(s * zr, zr)])

        plsc.subcore_barrier()

        def gather(chunk, buf, sem):
            pltpu.async_copy(g_sh.at[src_v.at[chunk]], buf, sem)

        def drain(buf, sem):
            pltpu.make_async_copy(g_sh.at[src_v.at[0]], buf, sem).wait()

        def scatter(chunk, buf):
            pltpu.sync_copy(buf, acc_sh.at[dst_v.at[chunk]], add=True)

        for b in range(nb):
            gather(b, bufs_a[b], sem_a)

        @pl.loop(0, k_per_w - 2 * nb, step=2 * nb)
        def _(jv):
            for b in range(nb):
                gather(jv + nb + b, bufs_b[b], sem_b)
            for b in range(nb):
                drain(bufs_a[b], sem_a)
            for b in range(nb):
                scatter(jv + b, bufs_a[b])
            for b in range(nb):
                gather(jv + 2 * nb + b, bufs_a[b], sem_a)
            for b in range(nb):
                drain(bufs_b[b], sem_b)
            for b in range(nb):
                scatter(jv + nb + b, bufs_b[b])

        last = k_per_w - 2 * nb
        for b in range(nb):
            gather(last + nb + b, bufs_b[b], sem_b)
        for b in range(nb):
            drain(bufs_a[b], sem_a)
        for b in range(nb):
            scatter(last + b, bufs_a[b])
        for b in range(nb):
            drain(bufs_b[b], sem_b)
        for b in range(nb):
            scatter(last + nb + b, bufs_b[b])

        plsc.subcore_barrier()

        @pl.loop(0, zr, step=CHUNK)
        def _(r):
            pltpu.sync_copy(acc_sh.at[pl.ds(s * zr + r, CHUNK)],
                            out_hbm.at[c, pl.ds(s * zr + r, CHUNK)])

    return k(g, srcm, dstm)


def kernel(x, edge_index, W1, b1, W2, b2):
    n, _ = x.shape
    e = edge_index.shape[1]
    npad = _round_up(n + 1, NS * CHUNK)
    epad = _round_up(e, NW * CHUNK * 8)

    src = edge_index[0].astype(jnp.int32)
    dst = edge_index[1].astype(jnp.int32)
    pad = jnp.full((epad - e,), n, jnp.int32)
    srcm = jnp.concatenate([src, pad]).reshape(-1, CHUNK)
    dstm = jnp.concatenate([dst, pad]).reshape(-1, CHUNK)
    x_p = jnp.zeros((npad, x.shape[1]), x.dtype).at[:n].set(x)

    h1 = _tc_matmul(x_p, W1)
    degp = _sc_degree(dstm, npad)
    dis, g1 = _tc_scale1(degp, h1)
    S1 = _sc_scatter(g1, srcm, dstm)
    g2 = _tc_mid(S1, g1, dis, b1)
    S2 = _sc_scatter(g2, srcm, dstm)
    out = _tc_final(S2, g2, dis, W2, b2)
    return out[:n]


# speedup vs baseline: 65.6815x; 1.1056x over previous
import functools

import jax
import jax.numpy as jnp
from jax import lax
from jax.experimental import pallas as pl
from jax.experimental.pallas import tpu as pltpu
from jax.experimental.pallas import tpu_sc as plsc

NC = 2
NS = 16
NW = NC * NS
CHUNK = 128
DH = 16
NB = 4


def _round_up(a, b):
    return (a + b - 1) // b * b


def _tc_matmul(x, W1, npad):
    n = x.shape[0]

    def body(x_ref, w_ref, o_ref):
        h = jnp.dot(x_ref[...], w_ref[...], preferred_element_type=jnp.float32)
        o_ref[pl.ds(0, n), :] = h
        o_ref[pl.ds(n, npad - n), :] = jnp.zeros((npad - n, DH), jnp.float32)

    return pl.pallas_call(
        body,
        out_shape=jax.ShapeDtypeStruct((npad, DH), jnp.float32),
    )(x, W1)


def _tc_dis(degp):
    npad = degp.shape[1]

    def body(deg_ref, dis_ref):
        dis_ref[...] = lax.rsqrt(deg_ref[0, :] + deg_ref[1, :] + 1.0)

    return pl.pallas_call(
        body,
        out_shape=jax.ShapeDtypeStruct((npad,), jnp.float32),
    )(degp)


def _tc_final(S2, dis, W2, b2):
    npad = dis.shape[0]
    dout = W2.shape[1]

    def body(s_ref, dis_ref, w2_ref, b2_ref, o_ref):
        agg = dis_ref[...][:, None] * (s_ref[0] + s_ref[1])
        z = jnp.dot(agg, w2_ref[...], preferred_element_type=jnp.float32)
        z = z + b2_ref[...][None, :]
        m = jnp.max(z, axis=1, keepdims=True)
        lse = m + jnp.log(jnp.sum(jnp.exp(z - m), axis=1, keepdims=True))
        o_ref[...] = z - lse

    return pl.pallas_call(
        body,
        out_shape=jax.ShapeDtypeStruct((npad, dout), jnp.float32),
    )(S2, dis, W2, b2)


def _sc_mesh():
    return plsc.VectorSubcoreMesh(core_axis_name="c", subcore_axis_name="s",
                                  num_cores=NC, num_subcores=NS)


_SC_PARAMS = pltpu.CompilerParams(use_tc_tiling_on_sc=False,
                                  needs_layout_passes=False)


def _sc_degree(dstm, npad):
    k_per_w = dstm.shape[0] // NW
    zr = npad // NS

    @functools.partial(
        pl.kernel,
        out_type=jax.ShapeDtypeStruct((NC, npad), jnp.float32),
        mesh=_sc_mesh(),
        scratch_types=[
            pltpu.VMEM((k_per_w, CHUNK), jnp.int32),
            pltpu.VMEM((CHUNK,), jnp.float32),
            pltpu.VMEM_SHARED((npad,), jnp.float32),
            pltpu.SemaphoreType.DMA,
        ],
        compiler_params=_SC_PARAMS,
    )
    def k(dst_hbm, out_hbm, dst_v, ones_v, acc_sh, sem):
        c = lax.axis_index("c")
        s = lax.axis_index("s")
        wid = c * NS + s

        @pl.loop(0, CHUNK, step=16)
        def _(i):
            ones_v[pl.ds(i, 16)] = jnp.zeros((16,), jnp.float32)

        @pl.loop(0, zr, step=CHUNK)
        def _(r):
            pltpu.sync_copy(ones_v, acc_sh.at[pl.ds(s * zr + r, CHUNK)])

        pltpu.sync_copy(dst_hbm.at[pl.ds(wid * k_per_w, k_per_w)], dst_v)

        @pl.loop(0, CHUNK, step=16)
        def _(i):
            ones_v[pl.ds(i, 16)] = jnp.ones((16,), jnp.float32)

        plsc.subcore_barrier()

        depth = 16

        @pl.loop(0, depth)
        def _(j):
            pltpu.async_copy(ones_v, acc_sh.at[dst_v.at[j]], sem, add=True)

        @pl.loop(depth, k_per_w)
        def _(j):
            pltpu.async_copy(ones_v, acc_sh.at[dst_v.at[j]], sem, add=True)
            pltpu.make_async_copy(ones_v, acc_sh.at[dst_v.at[0]], sem).wait()

        @pl.loop(0, depth)
        def _(j):
            pltpu.make_async_copy(ones_v, acc_sh.at[dst_v.at[0]], sem).wait()

        plsc.subcore_barrier()

        @pl.loop(0, zr, step=CHUNK)
        def _(r):
            pltpu.sync_copy(acc_sh.at[pl.ds(s * zr + r, CHUNK)],
                            out_hbm.at[c, pl.ds(s * zr + r, CHUNK)])

    return k(dstm)


def _edge_pipeline(g_sh, acc_sh, src_v, dst_v, bufs_a, bufs_b, sem_a, sem_b,
                   k_per_w):
    def gather(chunk, buf, sem):
        pltpu.async_copy(g_sh.at[src_v.at[chunk]], buf, sem)

    def drain(buf, sem):
        pltpu.make_async_copy(g_sh.at[src_v.at[0]], buf, sem).wait()

    def scatter(chunk, buf):
        pltpu.sync_copy(buf, acc_sh.at[dst_v.at[chunk]], add=True)

    for b in range(NB):
        gather(b, bufs_a[b], sem_a)

    @pl.loop(0, k_per_w - 2 * NB, step=2 * NB)
    def _(jv):
        for b in range(NB):
            gather(jv + NB + b, bufs_b[b], sem_b)
        for b in range(NB):
            drain(bufs_a[b], sem_a)
        for b in range(NB):
            scatter(jv + b, bufs_a[b])
        for b in range(NB):
            gather(jv + 2 * NB + b, bufs_a[b], sem_a)
        for b in range(NB):
            drain(bufs_b[b], sem_b)
        for b in range(NB):
            scatter(jv + NB + b, bufs_b[b])

    last = k_per_w - 2 * NB
    for b in range(NB):
        gather(last + NB + b, bufs_b[b], sem_b)
    for b in range(NB):
        drain(bufs_a[b], sem_a)
    for b in range(NB):
        scatter(last + b, bufs_a[b])
    for b in range(NB):
        drain(bufs_b[b], sem_b)
    for b in range(NB):
        scatter(last + NB + b, bufs_b[b])


def _scatter_scratch(k_per_w, npad, zr, extra_rows):
    return [
        pltpu.VMEM((k_per_w, CHUNK), jnp.int32),
        pltpu.VMEM((k_per_w, CHUNK), jnp.int32),
    ] + [pltpu.VMEM((CHUNK, DH), jnp.float32)] * (2 * NB) + [
        pltpu.VMEM((zr, DH), jnp.float32),
    ] * extra_rows + [
        pltpu.VMEM((zr,), jnp.float32),
        pltpu.VMEM_SHARED((npad, DH), jnp.float32),
        pltpu.VMEM_SHARED((npad, DH), jnp.float32),
        pltpu.SemaphoreType.DMA,
        pltpu.SemaphoreType.DMA,
    ]


def _stage_and_run(c, s, wid, k_per_w, zr, grows, src_hbm, dst_hbm, out_hbm,
                   src_v, dst_v, bufs_a, bufs_b, acc_sh, g_sh, sem_a, sem_b):
    base = s * zr
    pltpu.sync_copy(grows, g_sh.at[pl.ds(base, zr)])

    @pl.when(c == 0)
    def _():
        pltpu.sync_copy(grows, acc_sh.at[pl.ds(base, zr)])

    @pl.when(c != 0)
    def _():
        @pl.loop(0, CHUNK)
        def _(i):
            bufs_a[0][i] = jnp.zeros((DH,), jnp.float32)

        @pl.loop(0, zr, step=CHUNK)
        def _(r):
            pltpu.sync_copy(bufs_a[0], acc_sh.at[pl.ds(base + r, CHUNK)])

    pltpu.sync_copy(src_hbm.at[pl.ds(wid * k_per_w, k_per_w)], src_v)
    pltpu.sync_copy(dst_hbm.at[pl.ds(wid * k_per_w, k_per_w)], dst_v)

    plsc.subcore_barrier()

    _edge_pipeline(g_sh, acc_sh, src_v, dst_v, bufs_a, bufs_b, sem_a, sem_b,
                   k_per_w)

    plsc.subcore_barrier()

    @pl.loop(0, zr, step=CHUNK)
    def _(r):
        pltpu.sync_copy(acc_sh.at[pl.ds(base + r, CHUNK)],
                        out_hbm.at[c, pl.ds(base + r, CHUNK)])


def _sc_pass1(h1, dis, srcm, dstm):
    npad = h1.shape[0]
    k_per_w = srcm.shape[0] // NW
    zr = npad // NS

    @functools.partial(
        pl.kernel,
        out_type=jax.ShapeDtypeStruct((NC, npad, DH), jnp.float32),
        mesh=_sc_mesh(),
        scratch_types=_scatter_scratch(k_per_w, npad, zr, 1),
        compiler_params=_SC_PARAMS,
    )
    def k(h_hbm, dis_hbm, src_hbm, dst_hbm, out_hbm, src_v, dst_v, *rest):
        bufs_a = rest[:NB]
        bufs_b = rest[NB:2 * NB]
        hrows, disv, acc_sh, g_sh, sem_a, sem_b = rest[2 * NB:]
        c = lax.axis_index("c")
        s = lax.axis_index("s")
        wid = c * NS + s
        base = s * zr

        pltpu.sync_copy(h_hbm.at[pl.ds(base, zr)], hrows)
        pltpu.sync_copy(dis_hbm.at[pl.ds(base, zr)], disv)

        @pl.loop(0, zr)
        def _(i):
            d = plsc.load_gather(disv, [jnp.full((DH,), i, jnp.int32)])
            hrows[i] = hrows[i] * d

        _stage_and_run(c, s, wid, k_per_w, zr, hrows, src_hbm, dst_hbm,
                       out_hbm, src_v, dst_v, bufs_a, bufs_b, acc_sh, g_sh,
                       sem_a, sem_b)

    return k(h1, dis, srcm, dstm)


def _sc_pass2(S1, dis, b1, srcm, dstm):
    npad = S1.shape[1]
    k_per_w = srcm.shape[0] // NW
    zr = npad // NS

    @functools.partial(
        pl.kernel,
        out_type=jax.ShapeDtypeStruct((NC, npad, DH), jnp.float32),
        mesh=_sc_mesh(),
        scratch_types=[pltpu.VMEM((DH,), jnp.float32)]
        + _scatter_scratch(k_per_w, npad, zr, 2),
        compiler_params=_SC_PARAMS,
    )
    def k(s_hbm, dis_hbm, b1_hbm, src_hbm, dst_hbm, out_hbm, b1v, src_v,
          dst_v, *rest):
        bufs_a = rest[:NB]
        bufs_b = rest[NB:2 * NB]
        s0rows, s1rows, disv, acc_sh, g_sh, sem_a, sem_b = rest[2 * NB:]
        c = lax.axis_index("c")
        s = lax.axis_index("s")
        wid = c * NS + s
        base = s * zr

        pltpu.sync_copy(s_hbm.at[0, pl.ds(base, zr)], s0rows)
        pltpu.sync_copy(s_hbm.at[1, pl.ds(base, zr)], s1rows)
        pltpu.sync_copy(dis_hbm.at[pl.ds(base, zr)], disv)
        pltpu.sync_copy(b1_hbm, b1v)

        @pl.loop(0, zr)
        def _(i):
            d = plsc.load_gather(disv, [jnp.full((DH,), i, jnp.int32)])
            t = (s0rows[i] + s1rows[i]) * d + b1v[...]
            s0rows[i] = jnp.maximum(t, 0.0) * d

        _stage_and_run(c, s, wid, k_per_w, zr, s0rows, src_hbm, dst_hbm,
                       out_hbm, src_v, dst_v, bufs_a, bufs_b, acc_sh, g_sh,
                       sem_a, sem_b)

    return k(S1, dis, b1, srcm, dstm)


def kernel(x, edge_index, W1, b1, W2, b2):
    n, _ = x.shape
    e = edge_index.shape[1]
    npad = _round_up(n + 1, NS * CHUNK)
    epad = _round_up(e, NW * CHUNK * 8)

    src = edge_index[0].astype(jnp.int32)
    dst = edge_index[1].astype(jnp.int32)
    pad = jnp.full((epad - e,), n, jnp.int32)
    srcm = jnp.concatenate([src, pad]).reshape(-1, CHUNK)
    dstm = jnp.concatenate([dst, pad]).reshape(-1, CHUNK)

    h1 = _tc_matmul(x, W1, npad)
    degp = _sc_degree(dstm, npad)
    dis = _tc_dis(degp)
    S1 = _sc_pass1(h1, dis, srcm, dstm)
    S2 = _sc_pass2(S1, dis, b1, srcm, dstm)
    out = _tc_final(S2, dis, W2, b2)
    return out[:n]


# speedup vs baseline: 66.7920x; 1.0169x over previous
import functools

import jax
import jax.numpy as jnp
from jax import lax
from jax.experimental import pallas as pl
from jax.experimental.pallas import tpu as pltpu
from jax.experimental.pallas import tpu_sc as plsc

NC = 2
NS = 16
NW = NC * NS
CHUNK = 128
DH = 16
NB = 4


def _round_up(a, b):
    return (a + b - 1) // b * b


def _tc_matmul(x, W1, npad):
    n = x.shape[0]

    def body(x_ref, w_ref, o_ref):
        h = jnp.dot(x_ref[...], w_ref[...], preferred_element_type=jnp.float32)
        o_ref[pl.ds(0, n), :] = h
        o_ref[pl.ds(n, npad - n), :] = jnp.zeros((npad - n, DH), jnp.float32)

    return pl.pallas_call(
        body,
        out_shape=jax.ShapeDtypeStruct((npad, DH), jnp.float32),
    )(x, W1)


def _tc_dis(degp):
    npad = degp.shape[1]

    def body(deg_ref, dis_ref):
        dis_ref[...] = lax.rsqrt(deg_ref[0, :] + deg_ref[1, :] + 1.0)

    return pl.pallas_call(
        body,
        out_shape=jax.ShapeDtypeStruct((npad,), jnp.float32),
    )(degp)


def _tc_final(S2, dis, W2, b2, n):
    dout = W2.shape[1]
    W2e = jnp.tile(W2, (1, 128 // dout))
    W2s = jnp.tile(W2[:, ::-1], (1, 128 // dout))
    b2e = jnp.tile(b2, 128 // dout)
    b2s = jnp.tile(b2[::-1], 128 // dout)

    def body(s_ref, dis_ref, we_ref, ws_ref, be_ref, bs_ref, o_ref):
        agg = dis_ref[...][:, None] * (s_ref[0] + s_ref[1])
        z = jnp.dot(agg, we_ref[...], preferred_element_type=jnp.float32)
        zs = jnp.dot(agg, ws_ref[...], preferred_element_type=jnp.float32)
        z = z + be_ref[...][None, :]
        zs = zs + bs_ref[...][None, :]
        m = jnp.maximum(z, zs)
        out = z - m - jnp.log(jnp.exp(z - m) + jnp.exp(zs - m))
        o_ref[...] = out[:n, :dout]

    return pl.pallas_call(
        body,
        out_shape=jax.ShapeDtypeStruct((n, dout), jnp.float32),
    )(S2, dis, W2e, W2s, b2e, b2s)


def _sc_mesh():
    return plsc.VectorSubcoreMesh(core_axis_name="c", subcore_axis_name="s",
                                  num_cores=NC, num_subcores=NS)


_SC_PARAMS = pltpu.CompilerParams(use_tc_tiling_on_sc=False,
                                  needs_layout_passes=False)


def _sc_degree(dstm, npad):
    k_per_w = dstm.shape[0] // NW
    zr = npad // NS

    @functools.partial(
        pl.kernel,
        out_type=jax.ShapeDtypeStruct((NC, npad), jnp.float32),
        mesh=_sc_mesh(),
        scratch_types=[
            pltpu.VMEM((k_per_w, CHUNK), jnp.int32),
            pltpu.VMEM((CHUNK,), jnp.float32),
            pltpu.VMEM_SHARED((npad,), jnp.float32),
            pltpu.SemaphoreType.DMA,
        ],
        compiler_params=_SC_PARAMS,
    )
    def k(dst_hbm, out_hbm, dst_v, ones_v, acc_sh, sem):
        c = lax.axis_index("c")
        s = lax.axis_index("s")
        wid = c * NS + s

        @pl.loop(0, CHUNK, step=16)
        def _(i):
            ones_v[pl.ds(i, 16)] = jnp.zeros((16,), jnp.float32)

        @pl.loop(0, zr, step=CHUNK)
        def _(r):
            pltpu.sync_copy(ones_v, acc_sh.at[pl.ds(s * zr + r, CHUNK)])

        pltpu.sync_copy(dst_hbm.at[pl.ds(wid * k_per_w, k_per_w)], dst_v)

        @pl.loop(0, CHUNK, step=16)
        def _(i):
            ones_v[pl.ds(i, 16)] = jnp.ones((16,), jnp.float32)

        plsc.subcore_barrier()

        depth = 16

        @pl.loop(0, depth)
        def _(j):
            pltpu.async_copy(ones_v, acc_sh.at[dst_v.at[j]], sem, add=True)

        @pl.loop(depth, k_per_w)
        def _(j):
            pltpu.async_copy(ones_v, acc_sh.at[dst_v.at[j]], sem, add=True)
            pltpu.make_async_copy(ones_v, acc_sh.at[dst_v.at[0]], sem).wait()

        @pl.loop(0, depth)
        def _(j):
            pltpu.make_async_copy(ones_v, acc_sh.at[dst_v.at[0]], sem).wait()

        plsc.subcore_barrier()

        @pl.loop(0, zr, step=CHUNK)
        def _(r):
            pltpu.sync_copy(acc_sh.at[pl.ds(s * zr + r, CHUNK)],
                            out_hbm.at[c, pl.ds(s * zr + r, CHUNK)])

    return k(dstm)


def _edge_pipeline(g_sh, acc_sh, src_v, dst_v, bufs_a, bufs_b, sem_a, sem_b,
                   k_per_w):
    def gather(chunk, buf, sem):
        pltpu.async_copy(g_sh.at[src_v.at[chunk]], buf, sem)

    def drain(buf, sem):
        pltpu.make_async_copy(g_sh.at[src_v.at[0]], buf, sem).wait()

    def scatter(chunk, buf):
        pltpu.sync_copy(buf, acc_sh.at[dst_v.at[chunk]], add=True)

    for b in range(NB):
        gather(b, bufs_a[b], sem_a)

    @pl.loop(0, k_per_w - 2 * NB, step=2 * NB)
    def _(jv):
        for b in range(NB):
            gather(jv + NB + b, bufs_b[b], sem_b)
        for b in range(NB):
            drain(bufs_a[b], sem_a)
        for b in range(NB):
            scatter(jv + b, bufs_a[b])
        for b in range(NB):
            gather(jv + 2 * NB + b, bufs_a[b], sem_a)
        for b in range(NB):
            drain(bufs_b[b], sem_b)
        for b in range(NB):
            scatter(jv + NB + b, bufs_b[b])

    last = k_per_w - 2 * NB
    for b in range(NB):
        gather(last + NB + b, bufs_b[b], sem_b)
    for b in range(NB):
        drain(bufs_a[b], sem_a)
    for b in range(NB):
        scatter(last + b, bufs_a[b])
    for b in range(NB):
        drain(bufs_b[b], sem_b)
    for b in range(NB):
        scatter(last + NB + b, bufs_b[b])


def _scatter_scratch(k_per_w, npad, zr, extra_rows):
    return [
        pltpu.VMEM((k_per_w, CHUNK), jnp.int32),
        pltpu.VMEM((k_per_w, CHUNK), jnp.int32),
    ] + [pltpu.VMEM((CHUNK, DH), jnp.float32)] * (2 * NB) + [
        pltpu.VMEM((zr, DH), jnp.float32),
    ] * extra_rows + [
        pltpu.VMEM((zr,), jnp.float32),
        pltpu.VMEM_SHARED((npad, DH), jnp.float32),
        pltpu.VMEM_SHARED((npad, DH), jnp.float32),
        pltpu.SemaphoreType.DMA,
        pltpu.SemaphoreType.DMA,
    ]


def _stage_and_run(c, s, wid, k_per_w, zr, grows, src_hbm, dst_hbm, out_hbm,
                   src_v, dst_v, bufs_a, bufs_b, acc_sh, g_sh, sem_a, sem_b):
    base = s * zr
    pltpu.sync_copy(grows, g_sh.at[pl.ds(base, zr)])

    @pl.when(c == 0)
    def _():
        pltpu.sync_copy(grows, acc_sh.at[pl.ds(base, zr)])

    @pl.when(c != 0)
    def _():
        @pl.loop(0, CHUNK)
        def _(i):
            bufs_a[0][i] = jnp.zeros((DH,), jnp.float32)

        @pl.loop(0, zr, step=CHUNK)
        def _(r):
            pltpu.sync_copy(bufs_a[0], acc_sh.at[pl.ds(base + r, CHUNK)])

    pltpu.sync_copy(src_hbm.at[pl.ds(wid * k_per_w, k_per_w)], src_v)
    pltpu.sync_copy(dst_hbm.at[pl.ds(wid * k_per_w, k_per_w)], dst_v)

    plsc.subcore_barrier()

    _edge_pipeline(g_sh, acc_sh, src_v, dst_v, bufs_a, bufs_b, sem_a, sem_b,
                   k_per_w)

    plsc.subcore_barrier()

    @pl.loop(0, zr, step=CHUNK)
    def _(r):
        pltpu.sync_copy(acc_sh.at[pl.ds(base + r, CHUNK)],
                        out_hbm.at[c, pl.ds(base + r, CHUNK)])


def _sc_pass1(h1, dis, srcm, dstm):
    npad = h1.shape[0]
    k_per_w = srcm.shape[0] // NW
    zr = npad // NS

    @functools.partial(
        pl.kernel,
        out_type=jax.ShapeDtypeStruct((NC, npad, DH), jnp.float32),
        mesh=_sc_mesh(),
        scratch_types=_scatter_scratch(k_per_w, npad, zr, 1),
        compiler_params=_SC_PARAMS,
    )
    def k(h_hbm, dis_hbm, src_hbm, dst_hbm, out_hbm, src_v, dst_v, *rest):
        bufs_a = rest[:NB]
        bufs_b = rest[NB:2 * NB]
        hrows, disv, acc_sh, g_sh, sem_a, sem_b = rest[2 * NB:]
        c = lax.axis_index("c")
        s = lax.axis_index("s")
        wid = c * NS + s
        base = s * zr

        pltpu.sync_copy(h_hbm.at[pl.ds(base, zr)], hrows)
        pltpu.sync_copy(dis_hbm.at[pl.ds(base, zr)], disv)

        @pl.loop(0, zr)
        def _(i):
            d = plsc.load_gather(disv, [jnp.full((DH,), i, jnp.int32)])
            hrows[i] = hrows[i] * d

        _stage_and_run(c, s, wid, k_per_w, zr, hrows, src_hbm, dst_hbm,
                       out_hbm, src_v, dst_v, bufs_a, bufs_b, acc_sh, g_sh,
                       sem_a, sem_b)

    return k(h1, dis, srcm, dstm)


def _sc_pass2(S1, dis, b1, srcm, dstm):
    npad = S1.shape[1]
    k_per_w = srcm.shape[0] // NW
    zr = npad // NS

    @functools.partial(
        pl.kernel,
        out_type=jax.ShapeDtypeStruct((NC, npad, DH), jnp.float32),
        mesh=_sc_mesh(),
        scratch_types=[pltpu.VMEM((DH,), jnp.float32)]
        + _scatter_scratch(k_per_w, npad, zr, 2),
        compiler_params=_SC_PARAMS,
    )
    def k(s_hbm, dis_hbm, b1_hbm, src_hbm, dst_hbm, out_hbm, b1v, src_v,
          dst_v, *rest):
        bufs_a = rest[:NB]
        bufs_b = rest[NB:2 * NB]
        s0rows, s1rows, disv, acc_sh, g_sh, sem_a, sem_b = rest[2 * NB:]
        c = lax.axis_index("c")
        s = lax.axis_index("s")
        wid = c * NS + s
        base = s * zr

        pltpu.sync_copy(s_hbm.at[0, pl.ds(base, zr)], s0rows)
        pltpu.sync_copy(s_hbm.at[1, pl.ds(base, zr)], s1rows)
        pltpu.sync_copy(dis_hbm.at[pl.ds(base, zr)], disv)
        pltpu.sync_copy(b1_hbm, b1v)

        @pl.loop(0, zr)
        def _(i):
            d = plsc.load_gather(disv, [jnp.full((DH,), i, jnp.int32)])
            t = (s0rows[i] + s1rows[i]) * d + b1v[...]
            s0rows[i] = jnp.maximum(t, 0.0) * d

        _stage_and_run(c, s, wid, k_per_w, zr, s0rows, src_hbm, dst_hbm,
                       out_hbm, src_v, dst_v, bufs_a, bufs_b, acc_sh, g_sh,
                       sem_a, sem_b)

    return k(S1, dis, b1, srcm, dstm)


def kernel(x, edge_index, W1, b1, W2, b2):
    n, _ = x.shape
    e = edge_index.shape[1]
    npad = _round_up(n + 1, NS * CHUNK)
    epad = _round_up(e, NW * CHUNK * 8)

    src = edge_index[0].astype(jnp.int32)
    dst = edge_index[1].astype(jnp.int32)
    pad = jnp.full((epad - e,), n, jnp.int32)
    srcm = jnp.concatenate([src, pad]).reshape(-1, CHUNK)
    dstm = jnp.concatenate([dst, pad]).reshape(-1, CHUNK)

    h1 = _tc_matmul(x, W1, npad)
    degp = _sc_degree(dstm, npad)
    dis = _tc_dis(degp)
    S1 = _sc_pass1(h1, dis, srcm, dstm)
    S2 = _sc_pass2(S1, dis, b1, srcm, dstm)
    return _tc_final(S2, dis, W2, b2, n)


# speedup vs baseline: 67.4779x; 1.0103x over previous
import functools

import jax
import jax.numpy as jnp
from jax import lax
from jax.experimental import pallas as pl
from jax.experimental.pallas import tpu as pltpu
from jax.experimental.pallas import tpu_sc as plsc

NC = 2
NS = 16
NW = NC * NS
CHUNK = 128
DH = 16
NB = 4


def _round_up(a, b):
    return (a + b - 1) // b * b


def _tc_matmul(x, W1, npad):
    n = x.shape[0]

    def body(x_ref, w_ref, o_ref):
        h = jnp.dot(x_ref[...], w_ref[...], preferred_element_type=jnp.float32)
        o_ref[pl.ds(0, n), :] = h
        o_ref[pl.ds(n, npad - n), :] = jnp.zeros((npad - n, DH), jnp.float32)

    return pl.pallas_call(
        body,
        out_shape=jax.ShapeDtypeStruct((npad, DH), jnp.float32),
    )(x, W1)


def _tc_dis(degp):
    npad = degp.shape[1]

    def body(deg_ref, dis_ref):
        dis_ref[...] = lax.rsqrt(deg_ref[0, :] + deg_ref[1, :] + 1.0)

    return pl.pallas_call(
        body,
        out_shape=jax.ShapeDtypeStruct((npad,), jnp.float32),
    )(degp)


def _tc_final(S2, dis, W2, b2, n):
    dout = W2.shape[1]
    W2e = jnp.tile(W2, (1, 128 // dout))
    W2s = jnp.tile(W2[:, ::-1], (1, 128 // dout))
    b2e = jnp.tile(b2, 128 // dout)
    b2s = jnp.tile(b2[::-1], 128 // dout)

    def body(s_ref, dis_ref, we_ref, ws_ref, be_ref, bs_ref, o_ref):
        agg = dis_ref[...][:, None] * (s_ref[0] + s_ref[1])
        z = jnp.dot(agg, we_ref[...], preferred_element_type=jnp.float32)
        zs = jnp.dot(agg, ws_ref[...], preferred_element_type=jnp.float32)
        z = z + be_ref[...][None, :]
        zs = zs + bs_ref[...][None, :]
        m = jnp.maximum(z, zs)
        out = z - m - jnp.log(jnp.exp(z - m) + jnp.exp(zs - m))
        o_ref[...] = out[:n, :dout]

    return pl.pallas_call(
        body,
        out_shape=jax.ShapeDtypeStruct((n, dout), jnp.float32),
    )(S2, dis, W2e, W2s, b2e, b2s)


def _sc_mesh():
    return plsc.VectorSubcoreMesh(core_axis_name="c", subcore_axis_name="s",
                                  num_cores=NC, num_subcores=NS)


_SC_PARAMS = pltpu.CompilerParams(use_tc_tiling_on_sc=False,
                                  needs_layout_passes=False)


def _sc_degree(dstm, npad):
    k_per_w = dstm.shape[0] // NW
    zr = npad // NS

    @functools.partial(
        pl.kernel,
        out_type=jax.ShapeDtypeStruct((NC, npad), jnp.float32),
        mesh=_sc_mesh(),
        scratch_types=[
            pltpu.VMEM((k_per_w, CHUNK), jnp.int32),
            pltpu.VMEM((CHUNK,), jnp.float32),
            pltpu.VMEM_SHARED((npad,), jnp.float32),
            pltpu.SemaphoreType.DMA,
        ],
        compiler_params=_SC_PARAMS,
    )
    def k(dst_hbm, out_hbm, dst_v, ones_v, acc_sh, sem):
        c = lax.axis_index("c")
        s = lax.axis_index("s")
        wid = c * NS + s

        @pl.loop(0, CHUNK, step=16)
        def _(i):
            ones_v[pl.ds(i, 16)] = jnp.zeros((16,), jnp.float32)

        @pl.loop(0, zr, step=CHUNK)
        def _(r):
            pltpu.sync_copy(ones_v, acc_sh.at[pl.ds(s * zr + r, CHUNK)])

        pltpu.sync_copy(dst_hbm.at[pl.ds(wid * k_per_w, k_per_w)], dst_v)

        @pl.loop(0, CHUNK, step=16)
        def _(i):
            ones_v[pl.ds(i, 16)] = jnp.ones((16,), jnp.float32)

        plsc.subcore_barrier()

        depth = 16

        @pl.loop(0, depth)
        def _(j):
            pltpu.async_copy(ones_v, acc_sh.at[dst_v.at[j]], sem, add=True)

        @pl.loop(depth, k_per_w)
        def _(j):
            pltpu.async_copy(ones_v, acc_sh.at[dst_v.at[j]], sem, add=True)
            pltpu.make_async_copy(ones_v, acc_sh.at[dst_v.at[0]], sem).wait()

        @pl.loop(0, depth)
        def _(j):
            pltpu.make_async_copy(ones_v, acc_sh.at[dst_v.at[0]], sem).wait()

        plsc.subcore_barrier()

        @pl.loop(0, zr, step=CHUNK)
        def _(r):
            pltpu.sync_copy(acc_sh.at[pl.ds(s * zr + r, CHUNK)],
                            out_hbm.at[c, pl.ds(s * zr + r, CHUNK)])

    return k(dstm)


def _edge_pipeline(g_sh, acc_sh, src_v, dst_v, bufs_a, bufs_b, sem_ga, sem_gb,
                   sem_sa, sem_sb, k_per_w):
    def gather(chunk, buf, sem):
        pltpu.async_copy(g_sh.at[src_v.at[chunk]], buf, sem)

    def gdrain(buf, sem):
        pltpu.make_async_copy(g_sh.at[src_v.at[0]], buf, sem).wait()

    def scatter(chunk, buf, sem):
        pltpu.async_copy(buf, acc_sh.at[dst_v.at[chunk]], sem, add=True)

    def sdrain(buf, sem):
        pltpu.make_async_copy(buf, acc_sh.at[dst_v.at[0]], sem).wait()

    def gathers(jv, bufs, sem):
        for b in range(NB):
            gather(jv + b, bufs[b], sem)

    def gdrains(bufs, sem):
        for b in range(NB):
            gdrain(bufs[b], sem)

    def scatters(jv, bufs, sem):
        for b in range(NB):
            scatter(jv + b, bufs[b], sem)

    def sdrains(bufs, sem):
        for b in range(NB):
            sdrain(bufs[b], sem)

    gathers(0, bufs_a, sem_ga)
    gathers(NB, bufs_b, sem_gb)
    gdrains(bufs_a, sem_ga)
    scatters(0, bufs_a, sem_sa)
    sdrains(bufs_a, sem_sa)
    gathers(2 * NB, bufs_a, sem_ga)
    gdrains(bufs_b, sem_gb)
    scatters(NB, bufs_b, sem_sb)

    @pl.loop(2 * NB, k_per_w - 2 * NB, step=2 * NB)
    def _(jv):
        sdrains(bufs_b, sem_sb)
        gathers(jv + NB, bufs_b, sem_gb)
        gdrains(bufs_a, sem_ga)
        scatters(jv, bufs_a, sem_sa)
        sdrains(bufs_a, sem_sa)
        gathers(jv + 2 * NB, bufs_a, sem_ga)
        gdrains(bufs_b, sem_gb)
        scatters(jv + NB, bufs_b, sem_sb)

    last = k_per_w - 2 * NB
    sdrains(bufs_b, sem_sb)
    gathers(last + NB, bufs_b, sem_gb)
    gdrains(bufs_a, sem_ga)
    scatters(last, bufs_a, sem_sa)
    sdrains(bufs_a, sem_sa)
    gdrains(bufs_b, sem_gb)
    scatters(last + NB, bufs_b, sem_sb)
    sdrains(bufs_b, sem_sb)


def _scatter_scratch(k_per_w, npad, zr, extra_rows):
    return [
        pltpu.VMEM((k_per_w, CHUNK), jnp.int32),
        pltpu.VMEM((k_per_w, CHUNK), jnp.int32),
    ] + [pltpu.VMEM((CHUNK, DH), jnp.float32)] * (2 * NB) + [
        pltpu.VMEM((zr, DH), jnp.float32),
    ] * extra_rows + [
        pltpu.VMEM((zr,), jnp.float32),
        pltpu.VMEM_SHARED((npad, DH), jnp.float32),
        pltpu.VMEM_SHARED((npad, DH), jnp.float32),
    ] + [pltpu.SemaphoreType.DMA] * 4


def _stage_and_run(c, s, wid, k_per_w, zr, grows, src_hbm, dst_hbm, out_hbm,
                   src_v, dst_v, bufs_a, bufs_b, acc_sh, g_sh, sems):
    base = s * zr
    pltpu.sync_copy(grows, g_sh.at[pl.ds(base, zr)])

    @pl.when(c == 0)
    def _():
        pltpu.sync_copy(grows, acc_sh.at[pl.ds(base, zr)])

    @pl.when(c != 0)
    def _():
        @pl.loop(0, CHUNK)
        def _(i):
            bufs_a[0][i] = jnp.zeros((DH,), jnp.float32)

        @pl.loop(0, zr, step=CHUNK)
        def _(r):
            pltpu.sync_copy(bufs_a[0], acc_sh.at[pl.ds(base + r, CHUNK)])

    pltpu.sync_copy(src_hbm.at[pl.ds(wid * k_per_w, k_per_w)], src_v)
    pltpu.sync_copy(dst_hbm.at[pl.ds(wid * k_per_w, k_per_w)], dst_v)

    plsc.subcore_barrier()

    _edge_pipeline(g_sh, acc_sh, src_v, dst_v, bufs_a, bufs_b, *sems,
                   k_per_w=k_per_w)

    plsc.subcore_barrier()

    @pl.loop(0, zr, step=CHUNK)
    def _(r):
        pltpu.sync_copy(acc_sh.at[pl.ds(base + r, CHUNK)],
                        out_hbm.at[c, pl.ds(base + r, CHUNK)])


def _sc_pass1(h1, dis, srcm, dstm):
    npad = h1.shape[0]
    k_per_w = srcm.shape[0] // NW
    zr = npad // NS

    @functools.partial(
        pl.kernel,
        out_type=jax.ShapeDtypeStruct((NC, npad, DH), jnp.float32),
        mesh=_sc_mesh(),
        scratch_types=_scatter_scratch(k_per_w, npad, zr, 1),
        compiler_params=_SC_PARAMS,
    )
    def k(h_hbm, dis_hbm, src_hbm, dst_hbm, out_hbm, src_v, dst_v, *rest):
        bufs_a = rest[:NB]
        bufs_b = rest[NB:2 * NB]
        hrows, disv, acc_sh, g_sh = rest[2 * NB:2 * NB + 4]
        sems = rest[2 * NB + 4:]
        c = lax.axis_index("c")
        s = lax.axis_index("s")
        wid = c * NS + s
        base = s * zr

        pltpu.sync_copy(h_hbm.at[pl.ds(base, zr)], hrows)
        pltpu.sync_copy(dis_hbm.at[pl.ds(base, zr)], disv)

        @pl.loop(0, zr)
        def _(i):
            d = plsc.load_gather(disv, [jnp.full((DH,), i, jnp.int32)])
            hrows[i] = hrows[i] * d

        _stage_and_run(c, s, wid, k_per_w, zr, hrows, src_hbm, dst_hbm,
                       out_hbm, src_v, dst_v, bufs_a, bufs_b, acc_sh, g_sh,
                       sems)

    return k(h1, dis, srcm, dstm)


def _sc_pass2(S1, dis, b1, srcm, dstm):
    npad = S1.shape[1]
    k_per_w = srcm.shape[0] // NW
    zr = npad // NS

    @functools.partial(
        pl.kernel,
        out_type=jax.ShapeDtypeStruct((NC, npad, DH), jnp.float32),
        mesh=_sc_mesh(),
        scratch_types=[pltpu.VMEM((DH,), jnp.float32)]
        + _scatter_scratch(k_per_w, npad, zr, 2),
        compiler_params=_SC_PARAMS,
    )
    def k(s_hbm, dis_hbm, b1_hbm, src_hbm, dst_hbm, out_hbm, b1v, src_v,
          dst_v, *rest):
        bufs_a = rest[:NB]
        bufs_b = rest[NB:2 * NB]
        s0rows, s1rows, disv, acc_sh, g_sh = rest[2 * NB:2 * NB + 5]
        sems = rest[2 * NB + 5:]
        c = lax.axis_index("c")
        s = lax.axis_index("s")
        wid = c * NS + s
        base = s * zr

        pltpu.sync_copy(s_hbm.at[0, pl.ds(base, zr)], s0rows)
        pltpu.sync_copy(s_hbm.at[1, pl.ds(base, zr)], s1rows)
        pltpu.sync_copy(dis_hbm.at[pl.ds(base, zr)], disv)
        pltpu.sync_copy(b1_hbm, b1v)

        @pl.loop(0, zr)
        def _(i):
            d = plsc.load_gather(disv, [jnp.full((DH,), i, jnp.int32)])
            t = (s0rows[i] + s1rows[i]) * d + b1v[...]
            s0rows[i] = jnp.maximum(t, 0.0) * d

        _stage_and_run(c, s, wid, k_per_w, zr, s0rows, src_hbm, dst_hbm,
                       out_hbm, src_v, dst_v, bufs_a, bufs_b, acc_sh, g_sh,
                       sems)

    return k(S1, dis, b1, srcm, dstm)


def kernel(x, edge_index, W1, b1, W2, b2):
    n, _ = x.shape
    e = edge_index.shape[1]
    npad = _round_up(n + 1, NS * CHUNK)
    epad = _round_up(e, NW * CHUNK * 8)

    src = edge_index[0].astype(jnp.int32)
    dst = edge_index[1].astype(jnp.int32)
    pad = jnp.full((epad - e,), n, jnp.int32)
    srcm = jnp.concatenate([src, pad]).reshape(-1, CHUNK)
    dstm = jnp.concatenate([dst, pad]).reshape(-1, CHUNK)

    h1 = _tc_matmul(x, W1, npad)
    degp = _sc_degree(dstm, npad)
    dis = _tc_dis(degp)
    S1 = _sc_pass1(h1, dis, srcm, dstm)
    S2 = _sc_pass2(S1, dis, b1, srcm, dstm)
    return _tc_final(S2, dis, W2, b2, n)


# speedup vs baseline: 72.4337x; 1.0734x over previous
import functools

import jax
import jax.numpy as jnp
from jax import lax
from jax.experimental import pallas as pl
from jax.experimental.pallas import tpu as pltpu
from jax.experimental.pallas import tpu_sc as plsc

NC = 2
NS = 16
NW = NC * NS
CHUNK = 128
DH = 16
NB = 4


def _round_up(a, b):
    return (a + b - 1) // b * b


def _tc_matmul(x, W1, npad):
    n = x.shape[0]

    def body(x_ref, w_ref, o_ref):
        h = jnp.dot(x_ref[...], w_ref[...], preferred_element_type=jnp.float32)
        o_ref[pl.ds(0, n), :] = h
        o_ref[pl.ds(n, npad - n), :] = jnp.zeros((npad - n, DH), jnp.float32)

    return pl.pallas_call(
        body,
        out_shape=jax.ShapeDtypeStruct((npad, DH), jnp.float32),
    )(x, W1)


def _tc_dis(degp):
    npad = degp.shape[1]

    def body(deg_ref, dis_ref):
        dis_ref[...] = lax.rsqrt(deg_ref[0, :] + deg_ref[1, :] + 1.0)

    return pl.pallas_call(
        body,
        out_shape=jax.ShapeDtypeStruct((npad,), jnp.float32),
    )(degp)


def _tc_final(S2, dis, W2, b2, n):
    dout = W2.shape[1]
    W2e = jnp.tile(W2, (1, 128 // dout))
    W2s = jnp.tile(W2[:, ::-1], (1, 128 // dout))
    b2e = jnp.tile(b2, 128 // dout)
    b2s = jnp.tile(b2[::-1], 128 // dout)

    def body(s_ref, dis_ref, we_ref, ws_ref, be_ref, bs_ref, o_ref):
        agg = dis_ref[...][:, None] * (s_ref[0] + s_ref[1])
        z = jnp.dot(agg, we_ref[...], preferred_element_type=jnp.float32)
        zs = jnp.dot(agg, ws_ref[...], preferred_element_type=jnp.float32)
        z = z + be_ref[...][None, :]
        zs = zs + bs_ref[...][None, :]
        m = jnp.maximum(z, zs)
        out = z - m - jnp.log(jnp.exp(z - m) + jnp.exp(zs - m))
        o_ref[...] = out[:n, :dout]

    return pl.pallas_call(
        body,
        out_shape=jax.ShapeDtypeStruct((n, dout), jnp.float32),
    )(S2, dis, W2e, W2s, b2e, b2s)


def _sc_mesh():
    return plsc.VectorSubcoreMesh(core_axis_name="c", subcore_axis_name="s",
                                  num_cores=NC, num_subcores=NS)


_SC_PARAMS = pltpu.CompilerParams(use_tc_tiling_on_sc=False,
                                  needs_layout_passes=False)


def _sc_degree(em, npad):
    k_per_w = em.shape[1] // NW
    zr = npad // NS

    @functools.partial(
        pl.kernel,
        out_type=jax.ShapeDtypeStruct((NC, npad), jnp.float32),
        mesh=_sc_mesh(),
        scratch_types=[
            pltpu.VMEM((k_per_w, CHUNK), jnp.int32),
            pltpu.VMEM((CHUNK,), jnp.float32),
            pltpu.VMEM_SHARED((npad,), jnp.float32),
            pltpu.SemaphoreType.DMA,
        ],
        compiler_params=_SC_PARAMS,
    )
    def k(em_hbm, out_hbm, dst_v, ones_v, acc_sh, sem):
        c = lax.axis_index("c")
        s = lax.axis_index("s")
        wid = c * NS + s

        @pl.loop(0, CHUNK, step=16)
        def _(i):
            ones_v[pl.ds(i, 16)] = jnp.zeros((16,), jnp.float32)

        @pl.loop(0, zr, step=CHUNK)
        def _(r):
            pltpu.sync_copy(ones_v, acc_sh.at[pl.ds(s * zr + r, CHUNK)])

        pltpu.sync_copy(em_hbm.at[1, pl.ds(wid * k_per_w, k_per_w)], dst_v)

        @pl.loop(0, CHUNK, step=16)
        def _(i):
            ones_v[pl.ds(i, 16)] = jnp.ones((16,), jnp.float32)

        plsc.subcore_barrier()

        depth = 16

        @pl.loop(0, depth)
        def _(j):
            pltpu.async_copy(ones_v, acc_sh.at[dst_v.at[j]], sem, add=True)

        @pl.loop(depth, k_per_w)
        def _(j):
            pltpu.async_copy(ones_v, acc_sh.at[dst_v.at[j]], sem, add=True)
            pltpu.make_async_copy(ones_v, acc_sh.at[dst_v.at[0]], sem).wait()

        @pl.loop(0, depth)
        def _(j):
            pltpu.make_async_copy(ones_v, acc_sh.at[dst_v.at[0]], sem).wait()

        plsc.subcore_barrier()

        @pl.loop(0, zr, step=CHUNK)
        def _(r):
            pltpu.sync_copy(acc_sh.at[pl.ds(s * zr + r, CHUNK)],
                            out_hbm.at[c, pl.ds(s * zr + r, CHUNK)])

    return k(em)


def _edge_pipeline(g_sh, acc_sh, src_v, dst_v, bufs_a, bufs_b, sem_ga, sem_gb,
                   sem_sa, sem_sb, k_per_w):
    def gather(chunk, buf, sem):
        pltpu.async_copy(g_sh.at[src_v.at[chunk]], buf, sem)

    def gdrain(buf, sem):
        pltpu.make_async_copy(g_sh.at[src_v.at[0]], buf, sem).wait()

    def scatter(chunk, buf, sem):
        pltpu.async_copy(buf, acc_sh.at[dst_v.at[chunk]], sem, add=True)

    def sdrain(buf, sem):
        pltpu.make_async_copy(buf, acc_sh.at[dst_v.at[0]], sem).wait()

    def gathers(jv, bufs, sem):
        for b in range(NB):
            gather(jv + b, bufs[b], sem)

    def gdrains(bufs, sem):
        for b in range(NB):
            gdrain(bufs[b], sem)

    def scatters(jv, bufs, sem):
        for b in range(NB):
            scatter(jv + b, bufs[b], sem)

    def sdrains(bufs, sem):
        for b in range(NB):
            sdrain(bufs[b], sem)

    gathers(0, bufs_a, sem_ga)
    gathers(NB, bufs_b, sem_gb)
    gdrains(bufs_a, sem_ga)
    scatters(0, bufs_a, sem_sa)
    sdrains(bufs_a, sem_sa)
    gathers(2 * NB, bufs_a, sem_ga)
    gdrains(bufs_b, sem_gb)
    scatters(NB, bufs_b, sem_sb)

    @pl.loop(2 * NB, k_per_w - 2 * NB, step=2 * NB)
    def _(jv):
        sdrains(bufs_b, sem_sb)
        gathers(jv + NB, bufs_b, sem_gb)
        gdrains(bufs_a, sem_ga)
        scatters(jv, bufs_a, sem_sa)
        sdrains(bufs_a, sem_sa)
        gathers(jv + 2 * NB, bufs_a, sem_ga)
        gdrains(bufs_b, sem_gb)
        scatters(jv + NB, bufs_b, sem_sb)

    last = k_per_w - 2 * NB
    sdrains(bufs_b, sem_sb)
    gathers(last + NB, bufs_b, sem_gb)
    gdrains(bufs_a, sem_ga)
    scatters(last, bufs_a, sem_sa)
    sdrains(bufs_a, sem_sa)
    gdrains(bufs_b, sem_gb)
    scatters(last + NB, bufs_b, sem_sb)
    sdrains(bufs_b, sem_sb)


def _scatter_scratch(k_per_w, npad, zr, extra_rows):
    return [
        pltpu.VMEM((k_per_w, CHUNK), jnp.int32),
        pltpu.VMEM((k_per_w, CHUNK), jnp.int32),
    ] + [pltpu.VMEM((CHUNK, DH), jnp.float32)] * (2 * NB) + [
        pltpu.VMEM((zr, DH), jnp.float32),
    ] * extra_rows + [
        pltpu.VMEM((zr,), jnp.float32),
        pltpu.VMEM_SHARED((npad, DH), jnp.float32),
        pltpu.VMEM_SHARED((npad, DH), jnp.float32),
    ] + [pltpu.SemaphoreType.DMA] * 4


def _stage_and_run(c, s, wid, k_per_w, zr, grows, em_hbm, out_hbm,
                   src_v, dst_v, bufs_a, bufs_b, acc_sh, g_sh, sems):
    base = s * zr
    pltpu.sync_copy(grows, g_sh.at[pl.ds(base, zr)])

    @pl.when(c == 0)
    def _():
        pltpu.sync_copy(grows, acc_sh.at[pl.ds(base, zr)])

    @pl.when(c != 0)
    def _():
        @pl.loop(0, CHUNK)
        def _(i):
            bufs_a[0][i] = jnp.zeros((DH,), jnp.float32)

        @pl.loop(0, zr, step=CHUNK)
        def _(r):
            pltpu.sync_copy(bufs_a[0], acc_sh.at[pl.ds(base + r, CHUNK)])

    pltpu.sync_copy(em_hbm.at[0, pl.ds(wid * k_per_w, k_per_w)], src_v)
    pltpu.sync_copy(em_hbm.at[1, pl.ds(wid * k_per_w, k_per_w)], dst_v)

    plsc.subcore_barrier()

    _edge_pipeline(g_sh, acc_sh, src_v, dst_v, bufs_a, bufs_b, *sems,
                   k_per_w=k_per_w)

    plsc.subcore_barrier()

    @pl.loop(0, zr, step=CHUNK)
    def _(r):
        pltpu.sync_copy(acc_sh.at[pl.ds(base + r, CHUNK)],
                        out_hbm.at[c, pl.ds(base + r, CHUNK)])


def _sc_pass1(h1, dis, em):
    npad = h1.shape[0]
    k_per_w = em.shape[1] // NW
    zr = npad // NS

    @functools.partial(
        pl.kernel,
        out_type=jax.ShapeDtypeStruct((NC, npad, DH), jnp.float32),
        mesh=_sc_mesh(),
        scratch_types=_scatter_scratch(k_per_w, npad, zr, 1),
        compiler_params=_SC_PARAMS,
    )
    def k(h_hbm, dis_hbm, em_hbm, out_hbm, src_v, dst_v, *rest):
        bufs_a = rest[:NB]
        bufs_b = rest[NB:2 * NB]
        hrows, disv, acc_sh, g_sh = rest[2 * NB:2 * NB + 4]
        sems = rest[2 * NB + 4:]
        c = lax.axis_index("c")
        s = lax.axis_index("s")
        wid = c * NS + s
        base = s * zr

        pltpu.sync_copy(h_hbm.at[pl.ds(base, zr)], hrows)
        pltpu.sync_copy(dis_hbm.at[pl.ds(base, zr)], disv)

        @pl.loop(0, zr)
        def _(i):
            d = plsc.load_gather(disv, [jnp.full((DH,), i, jnp.int32)])
            hrows[i] = hrows[i] * d

        _stage_and_run(c, s, wid, k_per_w, zr, hrows, em_hbm,
                       out_hbm, src_v, dst_v, bufs_a, bufs_b, acc_sh, g_sh,
                       sems)

    return k(h1, dis, em)


def _sc_pass2(S1, dis, b1, em):
    npad = S1.shape[1]
    k_per_w = em.shape[1] // NW
    zr = npad // NS

    @functools.partial(
        pl.kernel,
        out_type=jax.ShapeDtypeStruct((NC, npad, DH), jnp.float32),
        mesh=_sc_mesh(),
        scratch_types=[pltpu.VMEM((DH,), jnp.float32)]
        + _scatter_scratch(k_per_w, npad, zr, 2),
        compiler_params=_SC_PARAMS,
    )
    def k(s_hbm, dis_hbm, b1_hbm, em_hbm, out_hbm, b1v, src_v,
          dst_v, *rest):
        bufs_a = rest[:NB]
        bufs_b = rest[NB:2 * NB]
        s0rows, s1rows, disv, acc_sh, g_sh = rest[2 * NB:2 * NB + 5]
        sems = rest[2 * NB + 5:]
        c = lax.axis_index("c")
        s = lax.axis_index("s")
        wid = c * NS + s
        base = s * zr

        pltpu.sync_copy(s_hbm.at[0, pl.ds(base, zr)], s0rows)
        pltpu.sync_copy(s_hbm.at[1, pl.ds(base, zr)], s1rows)
        pltpu.sync_copy(dis_hbm.at[pl.ds(base, zr)], disv)
        pltpu.sync_copy(b1_hbm, b1v)

        @pl.loop(0, zr)
        def _(i):
            d = plsc.load_gather(disv, [jnp.full((DH,), i, jnp.int32)])
            t = (s0rows[i] + s1rows[i]) * d + b1v[...]
            s0rows[i] = jnp.maximum(t, 0.0) * d

        _stage_and_run(c, s, wid, k_per_w, zr, s0rows, em_hbm,
                       out_hbm, src_v, dst_v, bufs_a, bufs_b, acc_sh, g_sh,
                       sems)

    return k(S1, dis, b1, em)


def kernel(x, edge_index, W1, b1, W2, b2):
    n, _ = x.shape
    e = edge_index.shape[1]
    npad = _round_up(n + 1, NS * CHUNK)
    epad = _round_up(e, NW * CHUNK * 8)

    em = jnp.reshape(edge_index.astype(jnp.int32), (2, e // CHUNK, CHUNK))
    em = jnp.pad(em, ((0, 0), (0, (epad - e) // CHUNK), (0, 0)),
                 constant_values=n)

    h1 = _tc_matmul(x, W1, npad)
    degp = _sc_degree(em, npad)
    dis = _tc_dis(degp)
    S1 = _sc_pass1(h1, dis, em)
    S2 = _sc_pass2(S1, dis, b1, em)
    return _tc_final(S2, dis, W2, b2, n)


# speedup vs baseline: 78.3854x; 1.0822x over previous
import functools

import jax
import jax.numpy as jnp
from jax import lax
from jax.experimental import pallas as pl
from jax.experimental.pallas import tpu as pltpu
from jax.experimental.pallas import tpu_sc as plsc

NC = 2
NS = 16
NW = NC * NS
CHUNK = 128
DH = 16
NB = 4


def _round_up(a, b):
    return (a + b - 1) // b * b


def _tc_matmul(x, W1, npad):
    n = x.shape[0]

    def body(x_ref, w_ref, o_ref):
        h = jnp.dot(x_ref[...], w_ref[...], preferred_element_type=jnp.float32)
        o_ref[pl.ds(0, n), :] = h
        o_ref[pl.ds(n, npad - n), :] = jnp.zeros((npad - n, DH), jnp.float32)

    return pl.pallas_call(
        body,
        out_shape=jax.ShapeDtypeStruct((npad, DH), jnp.float32),
    )(x, W1)


def _tc_dis(degp):
    npad = degp.shape[1]
    degf = jnp.reshape(degp, (NC * npad // 128, 128))
    half = degf.shape[0] // 2

    def body(deg_ref, dis_ref):
        d = deg_ref[...]
        dis_ref[...] = jnp.reshape(
            lax.rsqrt(d[:half] + d[half:] + 1.0), (npad,))

    return pl.pallas_call(
        body,
        out_shape=jax.ShapeDtypeStruct((npad,), jnp.float32),
    )(degf)


def _tc_final(S2, disp, W2, b2, n):
    npad, dout = disp.shape[0], W2.shape[1]
    s2f = jnp.reshape(S2, (NC * npad * DH // 128, 128))
    dispf = jnp.reshape(disp, (npad * DH // 128, 128))
    half = s2f.shape[0] // 2
    eye = jnp.eye(128 // DH, dtype=W2.dtype)
    W2k = jnp.kron(eye, W2)
    W2ks = jnp.kron(eye, W2[:, ::-1])
    b2k = jnp.tile(b2, 128 // DH)
    b2ks = jnp.tile(b2[::-1], 128 // DH)

    def body(s_ref, dp_ref, wk_ref, wks_ref, bk_ref, bks_ref, o_ref):
        s = s_ref[...]
        agg = dp_ref[...] * (s[:half] + s[half:])
        z = jnp.dot(agg, wk_ref[...], preferred_element_type=jnp.float32)
        zs = jnp.dot(agg, wks_ref[...], preferred_element_type=jnp.float32)
        z = z + bk_ref[...][None, :]
        zs = zs + bks_ref[...][None, :]
        m = jnp.maximum(z, zs)
        o_ref[...] = z - m - jnp.log(jnp.exp(z - m) + jnp.exp(zs - m))

    out_pk = pl.pallas_call(
        body,
        out_shape=jax.ShapeDtypeStruct((npad * DH // 128, DH), jnp.float32),
    )(s2f, dispf, W2k, W2ks, b2k, b2ks)
    return jnp.reshape(out_pk, (npad, dout))[:n, :]


def _sc_mesh():
    return plsc.VectorSubcoreMesh(core_axis_name="c", subcore_axis_name="s",
                                  num_cores=NC, num_subcores=NS)


_SC_PARAMS = pltpu.CompilerParams(use_tc_tiling_on_sc=False,
                                  needs_layout_passes=False)


def _sc_degree(em, npad):
    k_per_w = em.shape[1] // NW
    zr = npad // NS

    @functools.partial(
        pl.kernel,
        out_type=jax.ShapeDtypeStruct((NC, npad), jnp.float32),
        mesh=_sc_mesh(),
        scratch_types=[
            pltpu.VMEM((k_per_w, CHUNK), jnp.int32),
            pltpu.VMEM((CHUNK,), jnp.float32),
            pltpu.VMEM_SHARED((npad,), jnp.float32),
            pltpu.SemaphoreType.DMA,
        ],
        compiler_params=_SC_PARAMS,
    )
    def k(em_hbm, out_hbm, dst_v, ones_v, acc_sh, sem):
        c = lax.axis_index("c")
        s = lax.axis_index("s")
        wid = c * NS + s

        @pl.loop(0, CHUNK, step=16)
        def _(i):
            ones_v[pl.ds(i, 16)] = jnp.zeros((16,), jnp.float32)

        @pl.loop(0, zr, step=CHUNK)
        def _(r):
            pltpu.sync_copy(ones_v, acc_sh.at[pl.ds(s * zr + r, CHUNK)])

        pltpu.sync_copy(em_hbm.at[1, pl.ds(wid * k_per_w, k_per_w)], dst_v)

        @pl.loop(0, CHUNK, step=16)
        def _(i):
            ones_v[pl.ds(i, 16)] = jnp.ones((16,), jnp.float32)

        plsc.subcore_barrier()

        depth = 16

        @pl.loop(0, depth)
        def _(j):
            pltpu.async_copy(ones_v, acc_sh.at[dst_v.at[j]], sem, add=True)

        @pl.loop(depth, k_per_w)
        def _(j):
            pltpu.async_copy(ones_v, acc_sh.at[dst_v.at[j]], sem, add=True)
            pltpu.make_async_copy(ones_v, acc_sh.at[dst_v.at[0]], sem).wait()

        @pl.loop(0, depth)
        def _(j):
            pltpu.make_async_copy(ones_v, acc_sh.at[dst_v.at[0]], sem).wait()

        plsc.subcore_barrier()

        @pl.loop(0, zr, step=CHUNK)
        def _(r):
            pltpu.sync_copy(acc_sh.at[pl.ds(s * zr + r, CHUNK)],
                            out_hbm.at[c, pl.ds(s * zr + r, CHUNK)])

    return k(em)


def _edge_pipeline(g_sh, acc_sh, src_v, dst_v, bufs_a, bufs_b, sem_ga, sem_gb,
                   sem_sa, sem_sb, k_per_w):
    def gather(chunk, buf, sem):
        pltpu.async_copy(g_sh.at[src_v.at[chunk]], buf, sem)

    def gdrain(buf, sem):
        pltpu.make_async_copy(g_sh.at[src_v.at[0]], buf, sem).wait()

    def scatter(chunk, buf, sem):
        pltpu.async_copy(buf, acc_sh.at[dst_v.at[chunk]], sem, add=True)

    def sdrain(buf, sem):
        pltpu.make_async_copy(buf, acc_sh.at[dst_v.at[0]], sem).wait()

    def gathers(jv, bufs, sem):
        for b in range(NB):
            gather(jv + b, bufs[b], sem)

    def gdrains(bufs, sem):
        for b in range(NB):
            gdrain(bufs[b], sem)

    def scatters(jv, bufs, sem):
        for b in range(NB):
            scatter(jv + b, bufs[b], sem)

    def sdrains(bufs, sem):
        for b in range(NB):
            sdrain(bufs[b], sem)

    gathers(0, bufs_a, sem_ga)
    gathers(NB, bufs_b, sem_gb)
    gdrains(bufs_a, sem_ga)
    scatters(0, bufs_a, sem_sa)
    sdrains(bufs_a, sem_sa)
    gathers(2 * NB, bufs_a, sem_ga)
    gdrains(bufs_b, sem_gb)
    scatters(NB, bufs_b, sem_sb)

    @pl.loop(2 * NB, k_per_w - 2 * NB, step=2 * NB)
    def _(jv):
        sdrains(bufs_b, sem_sb)
        gathers(jv + NB, bufs_b, sem_gb)
        gdrains(bufs_a, sem_ga)
        scatters(jv, bufs_a, sem_sa)
        sdrains(bufs_a, sem_sa)
        gathers(jv + 2 * NB, bufs_a, sem_ga)
        gdrains(bufs_b, sem_gb)
        scatters(jv + NB, bufs_b, sem_sb)

    last = k_per_w - 2 * NB
    sdrains(bufs_b, sem_sb)
    gathers(last + NB, bufs_b, sem_gb)
    gdrains(bufs_a, sem_ga)
    scatters(last, bufs_a, sem_sa)
    sdrains(bufs_a, sem_sa)
    gdrains(bufs_b, sem_gb)
    scatters(last + NB, bufs_b, sem_sb)
    sdrains(bufs_b, sem_sb)


def _scatter_scratch(k_per_w, npad, zr, extra_rows):
    return [
        pltpu.VMEM((k_per_w, CHUNK), jnp.int32),
        pltpu.VMEM((k_per_w, CHUNK), jnp.int32),
    ] + [pltpu.VMEM((CHUNK, DH), jnp.float32)] * (2 * NB) + [
        pltpu.VMEM((zr, DH), jnp.float32),
    ] * extra_rows + [
        pltpu.VMEM((zr,), jnp.float32),
        pltpu.VMEM_SHARED((npad, DH), jnp.float32),
        pltpu.VMEM_SHARED((npad, DH), jnp.float32),
    ] + [pltpu.SemaphoreType.DMA] * 4


def _stage_and_run(c, s, wid, k_per_w, zr, grows, em_hbm, out_hbm,
                   src_v, dst_v, bufs_a, bufs_b, acc_sh, g_sh, sems):
    base = s * zr
    pltpu.sync_copy(grows, g_sh.at[pl.ds(base, zr)])

    @pl.when(c == 0)
    def _():
        pltpu.sync_copy(grows, acc_sh.at[pl.ds(base, zr)])

    @pl.when(c != 0)
    def _():
        @pl.loop(0, CHUNK)
        def _(i):
            bufs_a[0][i] = jnp.zeros((DH,), jnp.float32)

        @pl.loop(0, zr, step=CHUNK)
        def _(r):
            pltpu.sync_copy(bufs_a[0], acc_sh.at[pl.ds(base + r, CHUNK)])

    pltpu.sync_copy(em_hbm.at[0, pl.ds(wid * k_per_w, k_per_w)], src_v)
    pltpu.sync_copy(em_hbm.at[1, pl.ds(wid * k_per_w, k_per_w)], dst_v)

    plsc.subcore_barrier()

    _edge_pipeline(g_sh, acc_sh, src_v, dst_v, bufs_a, bufs_b, *sems,
                   k_per_w=k_per_w)

    plsc.subcore_barrier()

    @pl.loop(0, zr, step=CHUNK)
    def _(r):
        pltpu.sync_copy(acc_sh.at[pl.ds(base + r, CHUNK)],
                        out_hbm.at[c, pl.ds(base + r, CHUNK)])


def _sc_pass1(h1, dis, em):
    npad = h1.shape[0]
    k_per_w = em.shape[1] // NW
    zr = npad // NS

    @functools.partial(
        pl.kernel,
        out_type=jax.ShapeDtypeStruct((NC, npad, DH), jnp.float32),
        mesh=_sc_mesh(),
        scratch_types=_scatter_scratch(k_per_w, npad, zr, 1),
        compiler_params=_SC_PARAMS,
    )
    def k(h_hbm, dis_hbm, em_hbm, out_hbm, src_v, dst_v, *rest):
        bufs_a = rest[:NB]
        bufs_b = rest[NB:2 * NB]
        hrows, disv, acc_sh, g_sh = rest[2 * NB:2 * NB + 4]
        sems = rest[2 * NB + 4:]
        c = lax.axis_index("c")
        s = lax.axis_index("s")
        wid = c * NS + s
        base = s * zr

        pltpu.sync_copy(h_hbm.at[pl.ds(base, zr)], hrows)
        pltpu.sync_copy(dis_hbm.at[pl.ds(base, zr)], disv)

        @pl.loop(0, zr)
        def _(i):
            d = plsc.load_gather(disv, [jnp.full((DH,), i, jnp.int32)])
            hrows[i] = hrows[i] * d

        _stage_and_run(c, s, wid, k_per_w, zr, hrows, em_hbm,
                       out_hbm, src_v, dst_v, bufs_a, bufs_b, acc_sh, g_sh,
                       sems)

    return k(h1, dis, em)


def _sc_pass2(S1, dis, b1, em):
    npad = S1.shape[1]
    k_per_w = em.shape[1] // NW
    zr = npad // NS

    @functools.partial(
        pl.kernel,
        out_type=[
            jax.ShapeDtypeStruct((NC, npad, DH), jnp.float32),
            jax.ShapeDtypeStruct((npad, DH), jnp.float32),
        ],
        mesh=_sc_mesh(),
        scratch_types=[pltpu.VMEM((DH,), jnp.float32)]
        + _scatter_scratch(k_per_w, npad, zr, 3),
        compiler_params=_SC_PARAMS,
    )
    def k(s_hbm, dis_hbm, b1_hbm, em_hbm, out_hbm, disp_hbm, b1v, src_v,
          dst_v, *rest):
        bufs_a = rest[:NB]
        bufs_b = rest[NB:2 * NB]
        s0rows, s1rows, dispb, disv, acc_sh, g_sh = rest[2 * NB:2 * NB + 6]
        sems = rest[2 * NB + 6:]
        c = lax.axis_index("c")
        s = lax.axis_index("s")
        wid = c * NS + s
        base = s * zr

        pltpu.sync_copy(s_hbm.at[0, pl.ds(base, zr)], s0rows)
        pltpu.sync_copy(s_hbm.at[1, pl.ds(base, zr)], s1rows)
        pltpu.sync_copy(dis_hbm.at[pl.ds(base, zr)], disv)
        pltpu.sync_copy(b1_hbm, b1v)

        @pl.loop(0, zr)
        def _(i):
            d = plsc.load_gather(disv, [jnp.full((DH,), i, jnp.int32)])
            t = (s0rows[i] + s1rows[i]) * d + b1v[...]
            s0rows[i] = jnp.maximum(t, 0.0) * d
            dispb[i] = d

        @pl.when(c == 0)
        def _():
            pltpu.sync_copy(dispb, disp_hbm.at[pl.ds(base, zr)])

        _stage_and_run(c, s, wid, k_per_w, zr, s0rows, em_hbm,
                       out_hbm, src_v, dst_v, bufs_a, bufs_b, acc_sh, g_sh,
                       sems)

    return k(S1, dis, b1, em)


def kernel(x, edge_index, W1, b1, W2, b2):
    n, _ = x.shape
    e = edge_index.shape[1]
    npad = _round_up(n + 1, NS * CHUNK)
    epad = _round_up(e, NW * CHUNK * 8)

    em = jnp.reshape(edge_index.astype(jnp.int32), (2, e // CHUNK, CHUNK))
    em = jnp.pad(em, ((0, 0), (0, (epad - e) // CHUNK), (0, 0)),
                 constant_values=n)

    h1 = _tc_matmul(x, W1, npad)
    degp = _sc_degree(em, npad)
    dis = _tc_dis(degp)
    S1 = _sc_pass1(h1, dis, em)
    S2, disp = _sc_pass2(S1, dis, b1, em)
    return _tc_final(S2, disp, W2, b2, n)
